# baseline (speedup 1.0000x reference)
import functools

import jax
import jax.numpy as jnp
from jax import lax
from jax.experimental import pallas as pl
from jax.experimental.pallas import tpu as pltpu
from jax.experimental.pallas import tpu_sc as plsc

N = 10000
D = 128
E = 320000
T = 4

NW = 32
NPT = 314
NPAD = NW * NPT
SLOTS = 4 * NPT
NEG = -1e30

ROWB = 2000

_PERM = [(l % 2) * 64 + l // 2 for l in range(128)]


def _bf16_bits(v):
    import struct
    u = struct.unpack('<I', struct.pack('<f', v))[0]
    upper, lower = u >> 16, u & 0xFFFF
    if lower > 0x8000 or (lower == 0x8000 and (upper & 1)):
        upper += 1
    return upper & 0xFFFF


_NEGW_U = _bf16_bits(NEG) * 0x10001
NEGW = _NEGW_U - (1 << 32) if _NEGW_U >= (1 << 31) else _NEGW_U


def _pack_rows(h):
    ra = h[:, 0:64].astype(jnp.bfloat16).astype(jnp.float32)
    rb = h[:, 64:128].astype(jnp.bfloat16).astype(jnp.float32)
    ba = lax.bitcast_convert_type(ra, jnp.int32)
    bb = lax.bitcast_convert_type(rb, jnp.int32)
    return lax.bitwise_or(lax.shift_right_logical(ba, 16),
                          lax.bitwise_and(bb, jnp.int32(-65536)))
CH = 3200
NCH = E // CH
CAP = 3360


def _lstm_body(x_ref, wg_ref, bg_ref, h1_ref, h1w_ref):
    xs = jax.nn.sigmoid(x_ref[...])
    gates = jnp.dot(xs, wg_ref[...], preferred_element_type=jnp.float32) + bg_ref[...]
    i = gates[:, 0:D]
    g = gates[:, 2 * D:3 * D]
    o = gates[:, 3 * D:4 * D]
    c = jax.nn.sigmoid(i) * jnp.tanh(g)
    h = jax.nn.sigmoid(o) * jnp.tanh(c)
    h1_ref[...] = h
    h1w_ref[...] = _pack_rows(h)


def _lstm_stage(x, Wg, bg):
    return pl.pallas_call(
        _lstm_body,
        grid=(N // ROWB,),
        in_specs=[
            pl.BlockSpec((ROWB, D), lambda m: (m, 0)),
            pl.BlockSpec((D, 4 * D), lambda m: (0, 0)),
            pl.BlockSpec((1, 4 * D), lambda m: (0, 0)),
        ],
        out_specs=[
            pl.BlockSpec((ROWB, D), lambda m: (m, 0)),
            pl.BlockSpec((ROWB, 64), lambda m: (m, 0)),
        ],
        out_shape=[
            jax.ShapeDtypeStruct((N, D), jnp.float32),
            jax.ShapeDtypeStruct((N, 64), jnp.int32),
        ],
    )(x, Wg, bg)


def _pack_body(src_ref, dst_ref, et_ref, out_ref):
    s = src_ref[...]
    d = dst_ref[...]
    t = et_ref[...]
    out_ref[...] = lax.bitwise_or(lax.shift_left(d * 4 + t, 16), s)


def _pack_stage(src2, dst2, et2):
    rows = E // 128
    return pl.pallas_call(
        _pack_body,
        grid=(1,),
        in_specs=[pl.BlockSpec((rows, 128), lambda m: (0, 0))] * 3,
        out_specs=pl.BlockSpec((rows, 128), lambda m: (0, 0)),
        out_shape=jax.ShapeDtypeStruct((rows, 128), jnp.int32),
    )(src2, dst2, et2)


def _agg_term(a_ref, wc_ref):
    f = a_ref[...].astype(jnp.float32)
    f = jnp.where(f <= -1e29, 0.0, f)
    return jnp.dot(f, wc_ref[...], preferred_element_type=jnp.float32)


def _mix1_body(h1, a0, ws, wc0, bv, out_ref, st_ref):
    o = jnp.dot(h1[...], ws[...], preferred_element_type=jnp.float32)
    o += _agg_term(a0, wc0)
    o += bv[...]
    out_ref[...] = o
    s = jnp.concatenate([jnp.sum(o, axis=0)[None, :],
                         jnp.sum(o * o, axis=0)[None, :]], axis=0)

    @pl.when(pl.program_id(0) == 0)
    def _():
        st_ref[...] = s

    @pl.when(pl.program_id(0) != 0)
    def _():
        st_ref[...] += s


def _mix1_stage(h1, a0, ws, wc0, bv):
    return pl.pallas_call(
        _mix1_body,
        grid=(N // ROWB,),
        in_specs=[
            pl.BlockSpec((ROWB, D), lambda m: (m, 0)),
            pl.BlockSpec((ROWB, 512), lambda m: (m, 0)),
            pl.BlockSpec((D, 256), lambda m: (0, 0)),
            pl.BlockSpec((512, 256), lambda m: (0, 0)),
            pl.BlockSpec((1, 256), lambda m: (0, 0)),
        ],
        out_specs=[
            pl.BlockSpec((ROWB, 256), lambda m: (m, 0)),
            pl.BlockSpec((2, 256), lambda m: (0, 0)),
        ],
        out_shape=[
            jax.ShapeDtypeStruct((N, 256), jnp.float32),
            jax.ShapeDtypeStruct((2, 256), jnp.float32),
        ],
    )(h1, a0, ws, wc0, bv)


def _bnrelu_body(x_ref, st_ref, g_ref, b_ref, h_ref, w0_ref, w1_ref):
    st = st_ref[...]
    mu = st[0:1, :] / N
    var = st[1:2, :] / N - mu * mu
    scale = lax.rsqrt(var + 1e-5) * g_ref[...]
    h = jnp.maximum((x_ref[...] - mu) * scale + b_ref[...], 0.0)
    h_ref[...] = h
    w0_ref[...] = _pack_rows(h[:, 0:128])
    w1_ref[...] = _pack_rows(h[:, 128:256])


def _bnrelu_stage(x, st, gamma, beta):
    return pl.pallas_call(
        _bnrelu_body,
        grid=(N // ROWB,),
        in_specs=[
            pl.BlockSpec((ROWB, 256), lambda m: (m, 0)),
            pl.BlockSpec((2, 256), lambda m: (0, 0)),
            pl.BlockSpec((1, 256), lambda m: (0, 0)),
            pl.BlockSpec((1, 256), lambda m: (0, 0)),
        ],
        out_specs=[
            pl.BlockSpec((ROWB, 256), lambda m: (m, 0)),
            pl.BlockSpec((ROWB, 64), lambda m: (m, 0)),
            pl.BlockSpec((ROWB, 64), lambda m: (m, 0)),
        ],
        out_shape=[
            jax.ShapeDtypeStruct((N, 256), jnp.float32),
            jax.ShapeDtypeStruct((N, 64), jnp.int32),
            jax.ShapeDtypeStruct((N, 64), jnp.int32),
        ],
    )(x, st, gamma, beta)


def _mix2_body(h2, a0, a1, ws, wc0, wc1, bv, out_ref, st_ref):
    o = jnp.dot(h2[...], ws[...], preferred_element_type=jnp.float32)
    o += _agg_term(a0, wc0)
    o += _agg_term(a1, wc1)
    o += bv[...]
    out_ref[...] = o
    s = jnp.concatenate([jnp.sum(o, axis=0)[None, :],
                         jnp.sum(o * o, axis=0)[None, :]], axis=0)

    @pl.when(pl.program_id(0) == 0)
    def _():
        st_ref[...] = s

    @pl.when(pl.program_id(0) != 0)
    def _():
        st_ref[...] += s


def _mix2_stage(h2, a0, a1, ws, wc0, wc1, bv):
    return pl.pallas_call(
        _mix2_body,
        grid=(N // ROWB,),
        in_specs=[
            pl.BlockSpec((ROWB, 256), lambda m: (m, 0)),
            pl.BlockSpec((ROWB, 512), lambda m: (m, 0)),
            pl.BlockSpec((ROWB, 512), lambda m: (m, 0)),
            pl.BlockSpec((256, D), lambda m: (0, 0)),
            pl.BlockSpec((512, D), lambda m: (0, 0)),
            pl.BlockSpec((512, D), lambda m: (0, 0)),
            pl.BlockSpec((1, D), lambda m: (0, 0)),
        ],
        out_specs=[
            pl.BlockSpec((ROWB, D), lambda m: (m, 0)),
            pl.BlockSpec((2, D), lambda m: (0, 0)),
        ],
        out_shape=[
            jax.ShapeDtypeStruct((N, D), jnp.float32),
            jax.ShapeDtypeStruct((2, D), jnp.float32),
        ],
    )(h2, a0, a1, ws, wc0, wc1, bv)


def _final_body(x_ref, st_ref, g_ref, b_ref, out_ref):
    st = st_ref[...]
    mu = st[0:1, :] / N
    var = st[1:2, :] / N - mu * mu
    scale = lax.rsqrt(var + 1e-5) * g_ref[...]
    h = (x_ref[...] - mu) * scale + b_ref[...]
    out_ref[...] = jax.nn.sigmoid(h - 10.0)


def _final_stage(x, st, gamma, beta):
    return pl.pallas_call(
        _final_body,
        grid=(N // ROWB,),
        in_specs=[
            pl.BlockSpec((ROWB, D), lambda m: (m, 0)),
            pl.BlockSpec((2, D), lambda m: (0, 0)),
            pl.BlockSpec((1, D), lambda m: (0, 0)),
            pl.BlockSpec((1, D), lambda m: (0, 0)),
        ],
        out_specs=pl.BlockSpec((ROWB, D), lambda m: (m, 0)),
        out_shape=jax.ShapeDtypeStruct((N, D), jnp.float32),
    )(x, st, gamma, beta)


_SC_PARAMS = dict(
    compiler_params=pltpu.CompilerParams(needs_layout_passes=False,
                                         use_tc_tiling_on_sc=False),
)
LCH = 2048
ECAP = 158 * LCH
MAXCH = ECAP // LCH


def _sc_partition(packed):
    mesh = plsc.VectorSubcoreMesh(core_axis_name="c", subcore_axis_name="s")

    @functools.partial(
        pl.kernel,
        mesh=mesh,
        out_type=jax.ShapeDtypeStruct((NW * ECAP,), jnp.int32),
        scratch_types=[
            pltpu.VMEM((2 * CH,), jnp.int32),
            pltpu.VMEM((CAP,), jnp.int32),
            pltpu.SemaphoreType.DMA,
            pltpu.SemaphoreType.DMA,
        ],
        **_SC_PARAMS,
    )
    def k(packed_ref, lists, ebuf, pend, sema, semb):
        wid = lax.axis_index("s") * 2 + lax.axis_index("c")
        lo = wid * SLOTS
        hi = lo + SLOTS
        lbase = wid * ECAP
        iota = lax.iota(jnp.int32, 16)
        sentv = jnp.full((16,), lax.shift_left(hi, 16), jnp.int32)

        def initp(q, c):
            plsc.store_scatter(pend, [q * 16 + iota], sentv)
            return c
        lax.fori_loop(0, CAP // 16, initp, 0)

        def scan_flush(po, carry):
            pending, written = carry

            def step(j, pending):
                v = plsc.load_gather(ebuf, [po + j * 16 + iota])
                slot = lax.shift_right_logical(v, 16)
                mask = (slot >= lo) & (slot < hi)
                mi = mask.astype(jnp.int32)
                cs = plsc.cumsum(mi)
                cnt = jnp.sum(mi)
                pos = jnp.maximum(pending + cs - 1, 0)
                plsc.store_scatter(pend, [pos], v, mask=mask)
                return pending + cnt

            pending = lax.fori_loop(0, CH // 16, step, pending)

            nblk = lax.shift_right_logical(pending, 7)

            def wblk(b, c):
                pltpu.sync_copy(
                    pend.at[pl.ds(b * 128, 128)],
                    lists.at[pl.ds(lbase + (written + b) * 128, 128)])
                return c
            lax.fori_loop(0, nblk, wblk, 0)

            mbase = nblk * 128
            for g in range(8):
                rem = plsc.load_gather(pend, [mbase + g * 16 + iota])
                pend[g * 16:(g + 1) * 16] = rem
            return (lax.bitwise_and(pending, 127), written + nblk)

        def start_chunk(ci, half, sem):
            pltpu.async_copy(packed_ref.at[pl.ds(ci * CH, CH)],
                             ebuf.at[pl.ds(half * CH, CH)], sem)

        def wait_chunk(sem):
            pltpu.make_async_copy(packed_ref.at[pl.ds(0, CH)],
                                  ebuf.at[pl.ds(0, CH)], sem).wait()

        start_chunk(0, 0, sema)

        def pair_body(pp, carry):
            ci0 = pp * 2
            wait_chunk(sema)
            start_chunk(ci0 + 1, 1, semb)
            carry = scan_flush(0, carry)
            wait_chunk(semb)

            @pl.when(ci0 + 2 < NCH)
            def _():
                start_chunk(ci0 + 2, 0, sema)

            return scan_flush(CH, carry)

        pending, written = lax.fori_loop(0, NCH // 2, pair_body,
                                         (jnp.int32(0), jnp.int32(0)))

        def pads(q, c):
            plsc.store_scatter(pend, [pending + q * 16 + iota], sentv)
            return c
        lax.fori_loop(0, 17, pads, 0)
        ntail = lax.shift_right_logical(pending + 143, 7)

        def wtail(b, c):
            pltpu.sync_copy(
                pend.at[pl.ds(b * 128, 128)],
                lists.at[pl.ds(lbase + (written + b) * 128, 128)])
            return c
        lax.fori_loop(0, ntail, wtail, 0)
        written = written + ntail

        for q in range(8):
            pend[q * 16:(q + 1) * 16] = sentv

        def wsent(b, c):
            pltpu.sync_copy(
                pend.at[pl.ds(0, 128)],
                lists.at[pl.ds(lbase + (written + b) * 128, 128)])
            return c
        lax.fori_loop(0, LCH // 128, wsent, 0)

    return k(packed)


def _sc_agg(h_list, lists):
    nps = len(h_list)
    mesh = plsc.VectorSubcoreMesh(core_axis_name="c", subcore_axis_name="s")

    @functools.partial(
        pl.kernel,
        mesh=mesh,
        out_type=[jax.ShapeDtypeStruct((NPAD * 4 * 64,), jnp.int32)] * nps,
        scratch_types=[
            pltpu.VMEM((LCH,), jnp.int32),
            pltpu.VMEM((256,), jnp.int32),
            pltpu.VMEM((256, 64), jnp.int32),
            pltpu.VMEM(((SLOTS + 1) * 64,), jnp.int32),
            pltpu.SemaphoreType.DMA,
            pltpu.SemaphoreType.DMA,
        ],
        **_SC_PARAMS,
    )
    def k(*refs):
        h_refs = refs[:nps]
        lists_ref = refs[nps]
        out_refs = refs[nps + 1:nps + 1 + nps]
        ebuf, idxbuf, gbuf, acc, sema, semb = refs[nps + 1 + nps:]

        wid = lax.axis_index("s") * 2 + lax.axis_index("c")
        lo = wid * SLOTS
        hi = lo + SLOTS
        lbase = wid * ECAP
        iota = lax.iota(jnp.int32, 16)
        colv = [kk * 16 + iota for kk in range(4)]
        sent = lax.shift_left(hi, 16)
        negv = jnp.full((16,), NEGW, jnp.int32)

        for p in range(nps):
            h_hbm = h_refs[p]
            out_hbm = out_refs[p]

            def initb(j, c):
                plsc.store_scatter(acc, [j * 16 + iota], negv)
                return c
            lax.fori_loop(0, (SLOTS + 1) * 4, initb, 0)

            def apply_blk(b, po):
                def gloop(g, c1):
                    grp = plsc.load_gather(ebuf, [b * 128 + g * 16 + iota])
                    slotloc = lax.shift_right_logical(grp, 16) - lo

                    def rloop(r, c2):
                        rr = jnp.full((16,), r, jnp.int32)
                        sl64 = slotloc.at[rr].get(
                            mode="promise_in_bounds") * 64
                        row = jnp.full((16,), po + g * 16 + r, jnp.int32)
                        for kk in range(4):
                            msg = plsc.bitcast(
                                plsc.load_gather(gbuf, [row, colv[kk]]),
                                jnp.bfloat16)
                            idxk = sl64 + colv[kk]
                            cur = plsc.bitcast(
                                plsc.load_gather(acc, [idxk]), jnp.bfloat16)
                            plsc.store_scatter(
                                acc, [idxk],
                                plsc.bitcast(jnp.maximum(cur, msg),
                                             jnp.int32))
                        return c2
                    lax.fori_loop(0, 16, rloop, 0)
                    return c1
                lax.fori_loop(0, 8, gloop, 0)

            def build_start(b, hb, sem):
                bv = plsc.load_gather(ebuf, [b * 128 + iota])
                s0 = jnp.sum(jnp.where(iota == 0, bv, 0))
                bstart = s0 != sent

                def bloop(g, c1):
                    grp = plsc.load_gather(ebuf, [b * 128 + g * 16 + iota])
                    plsc.store_scatter(idxbuf, [hb + g * 16 + iota],
                                       lax.bitwise_and(grp, 0xFFFF))
                    return c1
                lax.fori_loop(0, 8, bloop, 0)

                @pl.when(bstart)
                def _():
                    pltpu.async_copy(
                        h_hbm.at[idxbuf.at[pl.ds(hb, 128)]],
                        gbuf.at[pl.ds(hb, 128)], sem)
                return bstart

            def wait_g(sem):
                pltpu.make_async_copy(
                    h_hbm.at[idxbuf.at[pl.ds(0, 128)]],
                    gbuf.at[pl.ds(0, 128)], sem).wait()

            def chunk_body(ci, go):
                running = go > 0

                @pl.when(running)
                def _():
                    pltpu.sync_copy(
                        lists_ref.at[pl.ds(lbase + ci * LCH, LCH)], ebuf)

                v0 = plsc.load_gather(ebuf, [iota])
                vl = plsc.load_gather(ebuf, [LCH - 16 + iota])
                s_first = jnp.sum(jnp.where(iota == 0, v0, 0))
                s_last = jnp.sum(jnp.where(iota == 15, vl, 0))
                process = running & (s_first != sent)
                go_next = running & (s_last != sent)

                def pair(pp, started_odd):
                    b0 = pp * 2
                    bs0 = build_start(b0, 0, sema)

                    @pl.when(started_odd > 0)
                    def _():
                        wait_g(semb)

                    @pl.when(pp > 0)
                    def _():
                        apply_blk(b0 - 1, 128)

                    bs1 = build_start(b0 + 1, 128, semb)

                    @pl.when(bs0)
                    def _():
                        wait_g(sema)

                    apply_blk(b0, 0)
                    return lax.select(bs1, 1, 0)

                npair = lax.select(process, LCH // 256, 0)
                started_odd = lax.fori_loop(0, npair, pair, jnp.int32(0))

                @pl.when(started_odd > 0)
                def _():
                    wait_g(semb)

                @pl.when(process)
                def _():
                    apply_blk(LCH // 128 - 1, 128)

                return lax.select(go_next, 1, 0)

            lax.fori_loop(0, MAXCH, chunk_body, jnp.int32(1))

            pltpu.sync_copy(acc.at[pl.ds(0, SLOTS * 64)],
                            out_hbm.at[pl.ds(wid * SLOTS * 64, SLOTS * 64)])

    return list(k(*h_list, lists))


def kernel(x, edge_index, edge_type, W_ih, W_hh, b_ih, b_hh,
           weights1, bias1, weights2, bias2,
           gamma1, beta1, gamma2, beta2,
           Wself1, bself1, Wself2, bself2, osc):
    Wg = W_ih.T
    bg = (b_ih + b_hh).reshape(1, 4 * D)
    perm = jnp.array(_PERM, jnp.int32)
    w1cp = weights1.transpose(0, 2, 1)[:, perm, :].reshape(512, 2 * D)
    b1 = (bself1 + 4.0 * bias1).reshape(1, 2 * D)
    w2t = weights2.transpose(0, 2, 1)
    w2cp = [w2t[:, 128 * c + perm, :].reshape(512, D) for c in range(2)]
    b2 = (bself2 + 4.0 * bias2).reshape(1, D)

    src2 = edge_index[0].reshape(E // 128, 128)
    dst2 = edge_index[1].reshape(E // 128, 128)
    et2 = edge_type.reshape(E // 128, 128)

    packed = _pack_stage(src2, dst2, et2).reshape(E)
    h1, h1w = _lstm_stage(x, Wg, bg)
    lists = _sc_partition(packed)

    def unpk(a):
        return lax.bitcast_convert_type(a, jnp.bfloat16).reshape(NPAD, 512)[:N]

    a1 = unpk(_sc_agg([h1w], lists)[0])

    out1, st1 = _mix1_stage(h1, a1, Wself1.T, w1cp, b1)
    h2, h2w0, h2w1 = _bnrelu_stage(out1, st1, gamma1.reshape(1, 2 * D),
                                   beta1.reshape(1, 2 * D))

    agg2 = _sc_agg([h2w0, h2w1], lists)
    a2 = [unpk(a) for a in agg2]

    out2, st2 = _mix2_stage(h2, a2[0], a2[1], Wself2.T, w2cp[0], w2cp[1], b2)
    return _final_stage(out2, st2, gamma2.reshape(1, D), beta2.reshape(1, D))


# speedup vs baseline: 4.4994x; 4.4994x over previous
import functools

import jax
import jax.numpy as jnp
from jax import lax
from jax.experimental import pallas as pl
from jax.experimental.pallas import tpu as pltpu
from jax.experimental.pallas import tpu_sc as plsc

N = 10000
D = 128
E = 320000
T = 4

NW = 32
NPT = 314
NPAD = NW * NPT
SLOTS = 4 * NPT
NEG = -1e30

ROWB = 2000

_PERM = [(l % 2) * 64 + l // 2 for l in range(128)]


def _bf16_bits(v):
    import struct
    u = struct.unpack('<I', struct.pack('<f', v))[0]
    upper, lower = u >> 16, u & 0xFFFF
    if lower > 0x8000 or (lower == 0x8000 and (upper & 1)):
        upper += 1
    return upper & 0xFFFF


_NEGW_U = _bf16_bits(NEG) * 0x10001
NEGW = _NEGW_U - (1 << 32) if _NEGW_U >= (1 << 31) else _NEGW_U


def _pack_rows(h):
    ra = h[:, 0:64].astype(jnp.bfloat16).astype(jnp.float32)
    rb = h[:, 64:128].astype(jnp.bfloat16).astype(jnp.float32)
    ba = lax.bitcast_convert_type(ra, jnp.int32)
    bb = lax.bitcast_convert_type(rb, jnp.int32)
    return lax.bitwise_or(lax.shift_right_logical(ba, 16),
                          lax.bitwise_and(bb, jnp.int32(-65536)))
CH = 3200
NCH = E // CH
CAP = 3360


def _lstm_body(x_ref, wg_ref, bg_ref, h1_ref, h1w_ref):
    xs = jax.nn.sigmoid(x_ref[...])
    gates = jnp.dot(xs, wg_ref[...], preferred_element_type=jnp.float32) + bg_ref[...]
    i = gates[:, 0:D]
    g = gates[:, 2 * D:3 * D]
    o = gates[:, 3 * D:4 * D]
    c = jax.nn.sigmoid(i) * jnp.tanh(g)
    h = jax.nn.sigmoid(o) * jnp.tanh(c)
    h1_ref[...] = h
    h1w_ref[...] = _pack_rows(h)


def _lstm_stage(x, Wg, bg):
    return pl.pallas_call(
        _lstm_body,
        grid=(N // ROWB,),
        in_specs=[
            pl.BlockSpec((ROWB, D), lambda m: (m, 0)),
            pl.BlockSpec((D, 4 * D), lambda m: (0, 0)),
            pl.BlockSpec((1, 4 * D), lambda m: (0, 0)),
        ],
        out_specs=[
            pl.BlockSpec((ROWB, D), lambda m: (m, 0)),
            pl.BlockSpec((ROWB, 64), lambda m: (m, 0)),
        ],
        out_shape=[
            jax.ShapeDtypeStruct((N, D), jnp.float32),
            jax.ShapeDtypeStruct((N, 64), jnp.int32),
        ],
    )(x, Wg, bg)


def _pack_body(src_ref, dst_ref, et_ref, out_ref):
    s = src_ref[...]
    d = dst_ref[...]
    t = et_ref[...]
    out_ref[...] = lax.bitwise_or(lax.shift_left(d * 4 + t, 16), s)


def _pack_stage(src2, dst2, et2):
    rows = E // 128
    return pl.pallas_call(
        _pack_body,
        grid=(1,),
        in_specs=[pl.BlockSpec((rows, 128), lambda m: (0, 0))] * 3,
        out_specs=pl.BlockSpec((rows, 128), lambda m: (0, 0)),
        out_shape=jax.ShapeDtypeStruct((rows, 128), jnp.int32),
    )(src2, dst2, et2)


def _agg_term(a_ref, wlo_ref, whi_ref):
    w = a_ref[...]
    flo = lax.bitcast_convert_type(lax.shift_left(w, 16), jnp.float32)
    fhi = lax.bitcast_convert_type(
        lax.bitwise_and(w, jnp.int32(-65536)), jnp.float32)
    flo = jnp.where(flo <= -1e29, 0.0, flo)
    fhi = jnp.where(fhi <= -1e29, 0.0, fhi)
    return (jnp.dot(flo, wlo_ref[...], preferred_element_type=jnp.float32)
            + jnp.dot(fhi, whi_ref[...], preferred_element_type=jnp.float32))


def _mix1_body(h1, a0, ws, wc0, wc1, bv, out_ref, st_ref):
    o = jnp.dot(h1[...], ws[...], preferred_element_type=jnp.float32)
    o += _agg_term(a0, wc0, wc1)
    o += bv[...]
    out_ref[...] = o
    s = jnp.concatenate([jnp.sum(o, axis=0)[None, :],
                         jnp.sum(o * o, axis=0)[None, :]], axis=0)

    @pl.when(pl.program_id(0) == 0)
    def _():
        st_ref[...] = s

    @pl.when(pl.program_id(0) != 0)
    def _():
        st_ref[...] += s


def _mix1_stage(h1, a0, ws, wc0, wc1, bv):
    return pl.pallas_call(
        _mix1_body,
        grid=(N // ROWB,),
        in_specs=[
            pl.BlockSpec((ROWB, D), lambda m: (m, 0)),
            pl.BlockSpec((ROWB, 256), lambda m: (m, 0)),
            pl.BlockSpec((D, 256), lambda m: (0, 0)),
            pl.BlockSpec((256, 256), lambda m: (0, 0)),
            pl.BlockSpec((256, 256), lambda m: (0, 0)),
            pl.BlockSpec((1, 256), lambda m: (0, 0)),
        ],
        out_specs=[
            pl.BlockSpec((ROWB, 256), lambda m: (m, 0)),
            pl.BlockSpec((2, 256), lambda m: (0, 0)),
        ],
        out_shape=[
            jax.ShapeDtypeStruct((N, 256), jnp.float32),
            jax.ShapeDtypeStruct((2, 256), jnp.float32),
        ],
    )(h1, a0, ws, wc0, wc1, bv)


def _bnrelu_body(x_ref, st_ref, g_ref, b_ref, h_ref, w0_ref, w1_ref):
    st = st_ref[...]
    mu = st[0:1, :] / N
    var = st[1:2, :] / N - mu * mu
    scale = lax.rsqrt(var + 1e-5) * g_ref[...]
    h = jnp.maximum((x_ref[...] - mu) * scale + b_ref[...], 0.0)
    h_ref[...] = h
    w0_ref[...] = _pack_rows(h[:, 0:128])
    w1_ref[...] = _pack_rows(h[:, 128:256])


def _bnrelu_stage(x, st, gamma, beta):
    return pl.pallas_call(
        _bnrelu_body,
        grid=(N // ROWB,),
        in_specs=[
            pl.BlockSpec((ROWB, 256), lambda m: (m, 0)),
            pl.BlockSpec((2, 256), lambda m: (0, 0)),
            pl.BlockSpec((1, 256), lambda m: (0, 0)),
            pl.BlockSpec((1, 256), lambda m: (0, 0)),
        ],
        out_specs=[
            pl.BlockSpec((ROWB, 256), lambda m: (m, 0)),
            pl.BlockSpec((ROWB, 64), lambda m: (m, 0)),
            pl.BlockSpec((ROWB, 64), lambda m: (m, 0)),
        ],
        out_shape=[
            jax.ShapeDtypeStruct((N, 256), jnp.float32),
            jax.ShapeDtypeStruct((N, 64), jnp.int32),
            jax.ShapeDtypeStruct((N, 64), jnp.int32),
        ],
    )(x, st, gamma, beta)


def _mix2_body(h2, a0, a1, ws, wc0, wc1, wc2, wc3, bv, out_ref, st_ref):
    o = jnp.dot(h2[...], ws[...], preferred_element_type=jnp.float32)
    o += _agg_term(a0, wc0, wc1)
    o += _agg_term(a1, wc2, wc3)
    o += bv[...]
    out_ref[...] = o
    s = jnp.concatenate([jnp.sum(o, axis=0)[None, :],
                         jnp.sum(o * o, axis=0)[None, :]], axis=0)

    @pl.when(pl.program_id(0) == 0)
    def _():
        st_ref[...] = s

    @pl.when(pl.program_id(0) != 0)
    def _():
        st_ref[...] += s


def _mix2_stage(h2, a0, a1, ws, wcs, bv):
    return pl.pallas_call(
        _mix2_body,
        grid=(N // ROWB,),
        in_specs=[
            pl.BlockSpec((ROWB, 256), lambda m: (m, 0)),
            pl.BlockSpec((ROWB, 256), lambda m: (m, 0)),
            pl.BlockSpec((ROWB, 256), lambda m: (m, 0)),
            pl.BlockSpec((256, D), lambda m: (0, 0)),
        ] + [pl.BlockSpec((256, D), lambda m: (0, 0))] * 4 + [
            pl.BlockSpec((1, D), lambda m: (0, 0)),
        ],
        out_specs=[
            pl.BlockSpec((ROWB, D), lambda m: (m, 0)),
            pl.BlockSpec((2, D), lambda m: (0, 0)),
        ],
        out_shape=[
            jax.ShapeDtypeStruct((N, D), jnp.float32),
            jax.ShapeDtypeStruct((2, D), jnp.float32),
        ],
    )(h2, a0, a1, ws, *wcs, bv)


def _final_body(x_ref, st_ref, g_ref, b_ref, out_ref):
    st = st_ref[...]
    mu = st[0:1, :] / N
    var = st[1:2, :] / N - mu * mu
    scale = lax.rsqrt(var + 1e-5) * g_ref[...]
    h = (x_ref[...] - mu) * scale + b_ref[...]
    out_ref[...] = jax.nn.sigmoid(h - 10.0)


def _final_stage(x, st, gamma, beta):
    return pl.pallas_call(
        _final_body,
        grid=(N // ROWB,),
        in_specs=[
            pl.BlockSpec((ROWB, D), lambda m: (m, 0)),
            pl.BlockSpec((2, D), lambda m: (0, 0)),
            pl.BlockSpec((1, D), lambda m: (0, 0)),
            pl.BlockSpec((1, D), lambda m: (0, 0)),
        ],
        out_specs=pl.BlockSpec((ROWB, D), lambda m: (m, 0)),
        out_shape=jax.ShapeDtypeStruct((N, D), jnp.float32),
    )(x, st, gamma, beta)


_SC_PARAMS = dict(
    compiler_params=pltpu.CompilerParams(needs_layout_passes=False,
                                         use_tc_tiling_on_sc=False),
)
LCH = 2048
ECAP = 158 * LCH
MAXCH = ECAP // LCH


def _sc_partition(packed):
    mesh = plsc.VectorSubcoreMesh(core_axis_name="c", subcore_axis_name="s")

    @functools.partial(
        pl.kernel,
        mesh=mesh,
        out_type=jax.ShapeDtypeStruct((NW * ECAP,), jnp.int32),
        scratch_types=[
            pltpu.VMEM((2 * CH,), jnp.int32),
            pltpu.VMEM((CAP,), jnp.int32),
            pltpu.SemaphoreType.DMA,
            pltpu.SemaphoreType.DMA,
        ],
        **_SC_PARAMS,
    )
    def k(packed_ref, lists, ebuf, pend, sema, semb):
        wid = lax.axis_index("s") * 2 + lax.axis_index("c")
        lo = wid * SLOTS
        hi = lo + SLOTS
        lbase = wid * ECAP
        iota = lax.iota(jnp.int32, 16)
        sentv = jnp.full((16,), lax.shift_left(hi, 16), jnp.int32)

        def initp(q, c):
            plsc.store_scatter(pend, [q * 16 + iota], sentv)
            return c
        lax.fori_loop(0, CAP // 16, initp, 0)

        def scan_flush(po, carry):
            pending, written = carry

            def step(j, pending):
                v = plsc.load_gather(ebuf, [po + j * 16 + iota])
                slot = lax.shift_right_logical(v, 16)
                mask = (slot >= lo) & (slot < hi)
                mi = mask.astype(jnp.int32)
                cs = plsc.cumsum(mi)
                cnt = jnp.sum(mi)
                pos = jnp.maximum(pending + cs - 1, 0)
                plsc.store_scatter(pend, [pos], v, mask=mask)
                return pending + cnt

            pending = lax.fori_loop(0, CH // 16, step, pending)

            nblk = lax.shift_right_logical(pending, 7)

            def wblk(b, c):
                pltpu.sync_copy(
                    pend.at[pl.ds(b * 128, 128)],
                    lists.at[pl.ds(lbase + (written + b) * 128, 128)])
                return c
            lax.fori_loop(0, nblk, wblk, 0)

            mbase = nblk * 128
            for g in range(8):
                rem = plsc.load_gather(pend, [mbase + g * 16 + iota])
                pend[g * 16:(g + 1) * 16] = rem
            return (lax.bitwise_and(pending, 127), written + nblk)

        def start_chunk(ci, half, sem):
            pltpu.async_copy(packed_ref.at[pl.ds(ci * CH, CH)],
                             ebuf.at[pl.ds(half * CH, CH)], sem)

        def wait_chunk(sem):
            pltpu.make_async_copy(packed_ref.at[pl.ds(0, CH)],
                                  ebuf.at[pl.ds(0, CH)], sem).wait()

        start_chunk(0, 0, sema)

        def pair_body(pp, carry):
            ci0 = pp * 2
            wait_chunk(sema)
            start_chunk(ci0 + 1, 1, semb)
            carry = scan_flush(0, carry)
            wait_chunk(semb)

            @pl.when(ci0 + 2 < NCH)
            def _():
                start_chunk(ci0 + 2, 0, sema)

            return scan_flush(CH, carry)

        pending, written = lax.fori_loop(0, NCH // 2, pair_body,
                                         (jnp.int32(0), jnp.int32(0)))

        def pads(q, c):
            plsc.store_scatter(pend, [pending + q * 16 + iota], sentv)
            return c
        lax.fori_loop(0, 17, pads, 0)
        ntail = lax.shift_right_logical(pending + 143, 7)

        def wtail(b, c):
            pltpu.sync_copy(
                pend.at[pl.ds(b * 128, 128)],
                lists.at[pl.ds(lbase + (written + b) * 128, 128)])
            return c
        lax.fori_loop(0, ntail, wtail, 0)
        written = written + ntail

        for q in range(8):
            pend[q * 16:(q + 1) * 16] = sentv

        def wsent(b, c):
            pltpu.sync_copy(
                pend.at[pl.ds(0, 128)],
                lists.at[pl.ds(lbase + (written + b) * 128, 128)])
            return c
        lax.fori_loop(0, LCH // 128, wsent, 0)

    return k(packed)


def _sc_agg(h_list, lists):
    nps = len(h_list)
    mesh = plsc.VectorSubcoreMesh(core_axis_name="c", subcore_axis_name="s")

    @functools.partial(
        pl.kernel,
        mesh=mesh,
        out_type=[jax.ShapeDtypeStruct((NPAD * 4 * 64,), jnp.int32)] * nps,
        scratch_types=[
            pltpu.VMEM((LCH,), jnp.int32),
            pltpu.VMEM((256,), jnp.int32),
            pltpu.VMEM((256, 64), jnp.int32),
            pltpu.VMEM(((SLOTS + 1) * 64,), jnp.int32),
            pltpu.SemaphoreType.DMA,
            pltpu.SemaphoreType.DMA,
        ],
        **_SC_PARAMS,
    )
    def k(*refs):
        h_refs = refs[:nps]
        lists_ref = refs[nps]
        out_refs = refs[nps + 1:nps + 1 + nps]
        ebuf, idxbuf, gbuf, acc, sema, semb = refs[nps + 1 + nps:]

        wid = lax.axis_index("s") * 2 + lax.axis_index("c")
        lo = wid * SLOTS
        hi = lo + SLOTS
        lbase = wid * ECAP
        iota = lax.iota(jnp.int32, 16)
        colv = [kk * 16 + iota for kk in range(4)]
        sent = lax.shift_left(hi, 16)
        negv = jnp.full((16,), NEGW, jnp.int32)

        for p in range(nps):
            h_hbm = h_refs[p]
            out_hbm = out_refs[p]

            def initb(j, c):
                plsc.store_scatter(acc, [j * 16 + iota], negv)
                return c
            lax.fori_loop(0, (SLOTS + 1) * 4, initb, 0)

            def apply_blk(b, po):
                def gloop(g, c1):
                    grp = plsc.load_gather(ebuf, [b * 128 + g * 16 + iota])
                    slotloc = lax.shift_right_logical(grp, 16) - lo

                    def rloop(r, c2):
                        rr = jnp.full((16,), r, jnp.int32)
                        sl64 = slotloc.at[rr].get(
                            mode="promise_in_bounds") * 64
                        row = jnp.full((16,), po + g * 16 + r, jnp.int32)
                        for kk in range(4):
                            msg = plsc.bitcast(
                                plsc.load_gather(gbuf, [row, colv[kk]]),
                                jnp.bfloat16)
                            idxk = sl64 + colv[kk]
                            cur = plsc.bitcast(
                                plsc.load_gather(acc, [idxk]), jnp.bfloat16)
                            plsc.store_scatter(
                                acc, [idxk],
                                plsc.bitcast(jnp.maximum(cur, msg),
                                             jnp.int32))
                        return c2
                    lax.fori_loop(0, 16, rloop, 0)
                    return c1
                lax.fori_loop(0, 8, gloop, 0)

            def build_start(b, hb, sem):
                bv = plsc.load_gather(ebuf, [b * 128 + iota])
                s0 = jnp.sum(jnp.where(iota == 0, bv, 0))
                bstart = s0 != sent

                def bloop(g, c1):
                    grp = plsc.load_gather(ebuf, [b * 128 + g * 16 + iota])
                    plsc.store_scatter(idxbuf, [hb + g * 16 + iota],
                                       lax.bitwise_and(grp, 0xFFFF))
                    return c1
                lax.fori_loop(0, 8, bloop, 0)

                @pl.when(bstart)
                def _():
                    pltpu.async_copy(
                        h_hbm.at[idxbuf.at[pl.ds(hb, 128)]],
                        gbuf.at[pl.ds(hb, 128)], sem)
                return bstart

            def wait_g(sem):
                pltpu.make_async_copy(
                    h_hbm.at[idxbuf.at[pl.ds(0, 128)]],
                    gbuf.at[pl.ds(0, 128)], sem).wait()

            def chunk_body(ci, go):
                running = go > 0

                @pl.when(running)
                def _():
                    pltpu.sync_copy(
                        lists_ref.at[pl.ds(lbase + ci * LCH, LCH)], ebuf)

                v0 = plsc.load_gather(ebuf, [iota])
                vl = plsc.load_gather(ebuf, [LCH - 16 + iota])
                s_first = jnp.sum(jnp.where(iota == 0, v0, 0))
                s_last = jnp.sum(jnp.where(iota == 15, vl, 0))
                process = running & (s_first != sent)
                go_next = running & (s_last != sent)

                def pair(pp, started_odd):
                    b0 = pp * 2
                    bs0 = build_start(b0, 0, sema)

                    @pl.when(started_odd > 0)
                    def _():
                        wait_g(semb)

                    @pl.when(pp > 0)
                    def _():
                        apply_blk(b0 - 1, 128)

                    bs1 = build_start(b0 + 1, 128, semb)

                    @pl.when(bs0)
                    def _():
                        wait_g(sema)

                    apply_blk(b0, 0)
                    return lax.select(bs1, 1, 0)

                npair = lax.select(process, LCH // 256, 0)
                started_odd = lax.fori_loop(0, npair, pair, jnp.int32(0))

                @pl.when(started_odd > 0)
                def _():
                    wait_g(semb)

                @pl.when(process)
                def _():
                    apply_blk(LCH // 128 - 1, 128)

                return lax.select(go_next, 1, 0)

            lax.fori_loop(0, MAXCH, chunk_body, jnp.int32(1))

            pltpu.sync_copy(acc.at[pl.ds(0, SLOTS * 64)],
                            out_hbm.at[pl.ds(wid * SLOTS * 64, SLOTS * 64)])

    return list(k(*h_list, lists))


def kernel(x, edge_index, edge_type, W_ih, W_hh, b_ih, b_hh,
           weights1, bias1, weights2, bias2,
           gamma1, beta1, gamma2, beta2,
           Wself1, bself1, Wself2, bself2, osc):
    Wg = W_ih.T
    bg = (b_ih + b_hh).reshape(1, 4 * D)
    w1c = [weights1[:, :, 64 * c:64 * (c + 1)].transpose(0, 2, 1).reshape(256, 2 * D)
           for c in range(2)]
    b1 = (bself1 + 4.0 * bias1).reshape(1, 2 * D)
    w2c = [weights2[:, :, 64 * c:64 * (c + 1)].transpose(0, 2, 1).reshape(256, D)
           for c in range(4)]
    b2 = (bself2 + 4.0 * bias2).reshape(1, D)

    src2 = edge_index[0].reshape(E // 128, 128)
    dst2 = edge_index[1].reshape(E // 128, 128)
    et2 = edge_type.reshape(E // 128, 128)

    packed = _pack_stage(src2, dst2, et2).reshape(E)
    h1, h1w = _lstm_stage(x, Wg, bg)
    lists = _sc_partition(packed)

    def unpk(a):
        return a.reshape(NPAD, 256)[:N]

    a1 = unpk(_sc_agg([h1w], lists)[0])

    out1, st1 = _mix1_stage(h1, a1, Wself1.T, w1c[0], w1c[1], b1)
    h2, h2w0, h2w1 = _bnrelu_stage(out1, st1, gamma1.reshape(1, 2 * D),
                                   beta1.reshape(1, 2 * D))

    agg2 = _sc_agg([h2w0, h2w1], lists)
    a2 = [unpk(a) for a in agg2]

    out2, st2 = _mix2_stage(h2, a2[0], a2[1], Wself2.T, w2c, b2)
    return _final_stage(out2, st2, gamma2.reshape(1, D), beta2.reshape(1, D))


# speedup vs baseline: 4.5089x; 1.0021x over previous
import functools

import jax
import jax.numpy as jnp
from jax import lax
from jax.experimental import pallas as pl
from jax.experimental.pallas import tpu as pltpu
from jax.experimental.pallas import tpu_sc as plsc

N = 10000
D = 128
E = 320000
T = 4

NW = 32
NPT = 314
NPAD = NW * NPT
SLOTS = 4 * NPT
NEG = -1e30

ROWB = 2000

_PERM = [(l % 2) * 64 + l // 2 for l in range(128)]


def _bf16_bits(v):
    import struct
    u = struct.unpack('<I', struct.pack('<f', v))[0]
    upper, lower = u >> 16, u & 0xFFFF
    if lower > 0x8000 or (lower == 0x8000 and (upper & 1)):
        upper += 1
    return upper & 0xFFFF


_NEGW_U = _bf16_bits(NEG) * 0x10001
NEGW = _NEGW_U - (1 << 32) if _NEGW_U >= (1 << 31) else _NEGW_U


def _pack_rows(h):
    ra = h[:, 0:64].astype(jnp.bfloat16).astype(jnp.float32)
    rb = h[:, 64:128].astype(jnp.bfloat16).astype(jnp.float32)
    ba = lax.bitcast_convert_type(ra, jnp.int32)
    bb = lax.bitcast_convert_type(rb, jnp.int32)
    return lax.bitwise_or(lax.shift_right_logical(ba, 16),
                          lax.bitwise_and(bb, jnp.int32(-65536)))
CH = 3200
NCH = E // CH
CAP = 3360


def _lstm_body(x_ref, wg_ref, bg_ref, h1_ref, h1w_ref):
    xs = jax.nn.sigmoid(x_ref[...])
    gates = jnp.dot(xs, wg_ref[...], preferred_element_type=jnp.float32) + bg_ref[...]
    i = gates[:, 0:D]
    g = gates[:, 2 * D:3 * D]
    o = gates[:, 3 * D:4 * D]
    c = jax.nn.sigmoid(i) * jnp.tanh(g)
    h = jax.nn.sigmoid(o) * jnp.tanh(c)
    h1_ref[...] = h
    h1w_ref[...] = _pack_rows(h)


def _lstm_stage(x, Wg, bg):
    return pl.pallas_call(
        _lstm_body,
        grid=(N // ROWB,),
        in_specs=[
            pl.BlockSpec((ROWB, D), lambda m: (m, 0)),
            pl.BlockSpec((D, 4 * D), lambda m: (0, 0)),
            pl.BlockSpec((1, 4 * D), lambda m: (0, 0)),
        ],
        out_specs=[
            pl.BlockSpec((ROWB, D), lambda m: (m, 0)),
            pl.BlockSpec((ROWB, 64), lambda m: (m, 0)),
        ],
        out_shape=[
            jax.ShapeDtypeStruct((N, D), jnp.float32),
            jax.ShapeDtypeStruct((N, 64), jnp.int32),
        ],
    )(x, Wg, bg)


def _pack_body(src_ref, dst_ref, et_ref, out_ref):
    s = src_ref[...]
    d = dst_ref[...]
    t = et_ref[...]
    out_ref[...] = lax.bitwise_or(lax.shift_left(d * 4 + t, 16), s)


def _pack_stage(src2, dst2, et2):
    rows = E // 128
    return pl.pallas_call(
        _pack_body,
        grid=(1,),
        in_specs=[pl.BlockSpec((rows, 128), lambda m: (0, 0))] * 3,
        out_specs=pl.BlockSpec((rows, 128), lambda m: (0, 0)),
        out_shape=jax.ShapeDtypeStruct((rows, 128), jnp.int32),
    )(src2, dst2, et2)


def _agg_term(a_ref, wlo_ref, whi_ref):
    w = a_ref[...]
    flo = lax.bitcast_convert_type(lax.shift_left(w, 16), jnp.float32)
    fhi = lax.bitcast_convert_type(
        lax.bitwise_and(w, jnp.int32(-65536)), jnp.float32)
    flo = jnp.where(flo <= -1e29, 0.0, flo)
    fhi = jnp.where(fhi <= -1e29, 0.0, fhi)
    return (jnp.dot(flo, wlo_ref[...], preferred_element_type=jnp.float32)
            + jnp.dot(fhi, whi_ref[...], preferred_element_type=jnp.float32))


def _mix1_body(h1, a0, ws, wc0, wc1, bv, out_ref, st_ref):
    o = jnp.dot(h1[...], ws[...], preferred_element_type=jnp.float32)
    o += _agg_term(a0, wc0, wc1)
    o += bv[...]
    out_ref[...] = o
    s = jnp.concatenate([jnp.sum(o, axis=0)[None, :],
                         jnp.sum(o * o, axis=0)[None, :]], axis=0)

    @pl.when(pl.program_id(0) == 0)
    def _():
        st_ref[...] = s

    @pl.when(pl.program_id(0) != 0)
    def _():
        st_ref[...] += s


def _mix1_stage(h1, a0, ws, wc0, wc1, bv):
    return pl.pallas_call(
        _mix1_body,
        grid=(N // ROWB,),
        in_specs=[
            pl.BlockSpec((ROWB, D), lambda m: (m, 0)),
            pl.BlockSpec((ROWB, 256), lambda m: (m, 0)),
            pl.BlockSpec((D, 256), lambda m: (0, 0)),
            pl.BlockSpec((256, 256), lambda m: (0, 0)),
            pl.BlockSpec((256, 256), lambda m: (0, 0)),
            pl.BlockSpec((1, 256), lambda m: (0, 0)),
        ],
        out_specs=[
            pl.BlockSpec((ROWB, 256), lambda m: (m, 0)),
            pl.BlockSpec((2, 256), lambda m: (0, 0)),
        ],
        out_shape=[
            jax.ShapeDtypeStruct((N, 256), jnp.float32),
            jax.ShapeDtypeStruct((2, 256), jnp.float32),
        ],
    )(h1, a0, ws, wc0, wc1, bv)


def _bnrelu_body(x_ref, st_ref, g_ref, b_ref, h_ref, w0_ref, w1_ref):
    st = st_ref[...]
    mu = st[0:1, :] / N
    var = st[1:2, :] / N - mu * mu
    scale = lax.rsqrt(var + 1e-5) * g_ref[...]
    h = jnp.maximum((x_ref[...] - mu) * scale + b_ref[...], 0.0)
    h_ref[...] = h
    w0_ref[...] = _pack_rows(h[:, 0:128])
    w1_ref[...] = _pack_rows(h[:, 128:256])


def _bnrelu_stage(x, st, gamma, beta):
    return pl.pallas_call(
        _bnrelu_body,
        grid=(N // ROWB,),
        in_specs=[
            pl.BlockSpec((ROWB, 256), lambda m: (m, 0)),
            pl.BlockSpec((2, 256), lambda m: (0, 0)),
            pl.BlockSpec((1, 256), lambda m: (0, 0)),
            pl.BlockSpec((1, 256), lambda m: (0, 0)),
        ],
        out_specs=[
            pl.BlockSpec((ROWB, 256), lambda m: (m, 0)),
            pl.BlockSpec((ROWB, 64), lambda m: (m, 0)),
            pl.BlockSpec((ROWB, 64), lambda m: (m, 0)),
        ],
        out_shape=[
            jax.ShapeDtypeStruct((N, 256), jnp.float32),
            jax.ShapeDtypeStruct((N, 64), jnp.int32),
            jax.ShapeDtypeStruct((N, 64), jnp.int32),
        ],
    )(x, st, gamma, beta)


def _mix2_body(h2, a0, a1, ws, wc0, wc1, wc2, wc3, bv, out_ref, st_ref):
    o = jnp.dot(h2[...], ws[...], preferred_element_type=jnp.float32)
    o += _agg_term(a0, wc0, wc1)
    o += _agg_term(a1, wc2, wc3)
    o += bv[...]
    out_ref[...] = o
    s = jnp.concatenate([jnp.sum(o, axis=0)[None, :],
                         jnp.sum(o * o, axis=0)[None, :]], axis=0)

    @pl.when(pl.program_id(0) == 0)
    def _():
        st_ref[...] = s

    @pl.when(pl.program_id(0) != 0)
    def _():
        st_ref[...] += s


def _mix2_stage(h2, a0, a1, ws, wcs, bv):
    return pl.pallas_call(
        _mix2_body,
        grid=(N // ROWB,),
        in_specs=[
            pl.BlockSpec((ROWB, 256), lambda m: (m, 0)),
            pl.BlockSpec((ROWB, 256), lambda m: (m, 0)),
            pl.BlockSpec((ROWB, 256), lambda m: (m, 0)),
            pl.BlockSpec((256, D), lambda m: (0, 0)),
        ] + [pl.BlockSpec((256, D), lambda m: (0, 0))] * 4 + [
            pl.BlockSpec((1, D), lambda m: (0, 0)),
        ],
        out_specs=[
            pl.BlockSpec((ROWB, D), lambda m: (m, 0)),
            pl.BlockSpec((2, D), lambda m: (0, 0)),
        ],
        out_shape=[
            jax.ShapeDtypeStruct((N, D), jnp.float32),
            jax.ShapeDtypeStruct((2, D), jnp.float32),
        ],
    )(h2, a0, a1, ws, *wcs, bv)


def _final_body(x_ref, st_ref, g_ref, b_ref, out_ref):
    st = st_ref[...]
    mu = st[0:1, :] / N
    var = st[1:2, :] / N - mu * mu
    scale = lax.rsqrt(var + 1e-5) * g_ref[...]
    h = (x_ref[...] - mu) * scale + b_ref[...]
    out_ref[...] = jax.nn.sigmoid(h - 10.0)


def _final_stage(x, st, gamma, beta):
    return pl.pallas_call(
        _final_body,
        grid=(N // ROWB,),
        in_specs=[
            pl.BlockSpec((ROWB, D), lambda m: (m, 0)),
            pl.BlockSpec((2, D), lambda m: (0, 0)),
            pl.BlockSpec((1, D), lambda m: (0, 0)),
            pl.BlockSpec((1, D), lambda m: (0, 0)),
        ],
        out_specs=pl.BlockSpec((ROWB, D), lambda m: (m, 0)),
        out_shape=jax.ShapeDtypeStruct((N, D), jnp.float32),
    )(x, st, gamma, beta)


_SC_PARAMS = dict(
    compiler_params=pltpu.CompilerParams(needs_layout_passes=False,
                                         use_tc_tiling_on_sc=False),
)
LCH = 2048
ECAP = 158 * LCH
MAXCH = ECAP // LCH


def _sc_partition(packed):
    mesh = plsc.VectorSubcoreMesh(core_axis_name="c", subcore_axis_name="s")

    @functools.partial(
        pl.kernel,
        mesh=mesh,
        out_type=jax.ShapeDtypeStruct((NW * ECAP,), jnp.int32),
        scratch_types=[
            pltpu.VMEM((2 * CH,), jnp.int32),
            pltpu.VMEM((CAP,), jnp.int32),
            pltpu.SemaphoreType.DMA,
            pltpu.SemaphoreType.DMA,
        ],
        **_SC_PARAMS,
    )
    def k(packed_ref, lists, ebuf, pend, sema, semb):
        wid = lax.axis_index("s") * 2 + lax.axis_index("c")
        lo = wid * SLOTS
        hi = lo + SLOTS
        lbase = wid * ECAP
        iota = lax.iota(jnp.int32, 16)
        sentv = jnp.full((16,), lax.shift_left(hi, 16), jnp.int32)

        def initp(q, c):
            plsc.store_scatter(pend, [q * 16 + iota], sentv)
            return c
        lax.fori_loop(0, CAP // 16, initp, 0)

        def scan_flush(po, carry):
            pending, written = carry

            def step(j, pending):
                v = plsc.load_gather(ebuf, [po + j * 16 + iota])
                slot = lax.shift_right_logical(v, 16)
                mask = (slot >= lo) & (slot < hi)
                mi = mask.astype(jnp.int32)
                cs = plsc.cumsum(mi)
                cnt = jnp.sum(mi)
                pos = jnp.maximum(pending + cs - 1, 0)
                plsc.store_scatter(pend, [pos], v, mask=mask)
                return pending + cnt

            pending = lax.fori_loop(0, CH // 16, step, pending)

            nblk = lax.shift_right_logical(pending, 7)

            def wblk(b, c):
                pltpu.sync_copy(
                    pend.at[pl.ds(b * 128, 128)],
                    lists.at[pl.ds(lbase + (written + b) * 128, 128)])
                return c
            lax.fori_loop(0, nblk, wblk, 0)

            mbase = nblk * 128
            for g in range(8):
                rem = plsc.load_gather(pend, [mbase + g * 16 + iota])
                pend[g * 16:(g + 1) * 16] = rem
            return (lax.bitwise_and(pending, 127), written + nblk)

        def start_chunk(ci, half, sem):
            pltpu.async_copy(packed_ref.at[pl.ds(ci * CH, CH)],
                             ebuf.at[pl.ds(half * CH, CH)], sem)

        def wait_chunk(sem):
            pltpu.make_async_copy(packed_ref.at[pl.ds(0, CH)],
                                  ebuf.at[pl.ds(0, CH)], sem).wait()

        start_chunk(0, 0, sema)

        def pair_body(pp, carry):
            ci0 = pp * 2
            wait_chunk(sema)
            start_chunk(ci0 + 1, 1, semb)
            carry = scan_flush(0, carry)
            wait_chunk(semb)

            @pl.when(ci0 + 2 < NCH)
            def _():
                start_chunk(ci0 + 2, 0, sema)

            return scan_flush(CH, carry)

        pending, written = lax.fori_loop(0, NCH // 2, pair_body,
                                         (jnp.int32(0), jnp.int32(0)))

        def pads(q, c):
            plsc.store_scatter(pend, [pending + q * 16 + iota], sentv)
            return c
        lax.fori_loop(0, 17, pads, 0)
        ntail = lax.shift_right_logical(pending + 143, 7)

        def wtail(b, c):
            pltpu.sync_copy(
                pend.at[pl.ds(b * 128, 128)],
                lists.at[pl.ds(lbase + (written + b) * 128, 128)])
            return c
        lax.fori_loop(0, ntail, wtail, 0)
        written = written + ntail

        for q in range(8):
            pend[q * 16:(q + 1) * 16] = sentv

        def wsent(b, c):
            pltpu.sync_copy(
                pend.at[pl.ds(0, 128)],
                lists.at[pl.ds(lbase + (written + b) * 128, 128)])
            return c
        lax.fori_loop(0, LCH // 128, wsent, 0)

    return k(packed)


def _sc_agg(h_list, lists):
    nps = len(h_list)
    mesh = plsc.VectorSubcoreMesh(core_axis_name="c", subcore_axis_name="s")

    @functools.partial(
        pl.kernel,
        mesh=mesh,
        out_type=[jax.ShapeDtypeStruct((NPAD * 4 * 64,), jnp.int32)] * nps,
        scratch_types=[
            pltpu.VMEM((LCH,), jnp.int32),
            pltpu.VMEM((512,), jnp.int32),
            pltpu.VMEM((512, 64), jnp.int32),
            pltpu.VMEM(((SLOTS + 1) * 64,), jnp.int32),
            pltpu.SemaphoreType.DMA,
            pltpu.SemaphoreType.DMA,
            pltpu.SemaphoreType.DMA,
            pltpu.SemaphoreType.DMA,
        ],
        **_SC_PARAMS,
    )
    def k(*refs):
        h_refs = refs[:nps]
        lists_ref = refs[nps]
        out_refs = refs[nps + 1:nps + 1 + nps]
        ebuf, idxbuf, gbuf, acc = refs[nps + 1 + nps:nps + 5 + nps]
        sems = refs[nps + 5 + nps:]

        wid = lax.axis_index("s") * 2 + lax.axis_index("c")
        lo = wid * SLOTS
        hi = lo + SLOTS
        lbase = wid * ECAP
        iota = lax.iota(jnp.int32, 16)
        colv = [kk * 16 + iota for kk in range(4)]
        sent = lax.shift_left(hi, 16)
        negv = jnp.full((16,), NEGW, jnp.int32)

        for p in range(nps):
            h_hbm = h_refs[p]
            out_hbm = out_refs[p]

            def initb(j, c):
                plsc.store_scatter(acc, [j * 16 + iota], negv)
                return c
            lax.fori_loop(0, (SLOTS + 1) * 4, initb, 0)

            def apply_blk(b, po):
                def gloop(g, c1):
                    grp = plsc.load_gather(ebuf, [b * 128 + g * 16 + iota])
                    slotloc = lax.shift_right_logical(grp, 16) - lo

                    def rloop(r, c2):
                        rr = jnp.full((16,), r, jnp.int32)
                        sl64 = slotloc.at[rr].get(
                            mode="promise_in_bounds") * 64
                        row = jnp.full((16,), po + g * 16 + r, jnp.int32)
                        for kk in range(4):
                            msg = plsc.bitcast(
                                plsc.load_gather(gbuf, [row, colv[kk]]),
                                jnp.bfloat16)
                            idxk = sl64 + colv[kk]
                            cur = plsc.bitcast(
                                plsc.load_gather(acc, [idxk]), jnp.bfloat16)
                            plsc.store_scatter(
                                acc, [idxk],
                                plsc.bitcast(jnp.maximum(cur, msg),
                                             jnp.int32))
                        return c2
                    lax.fori_loop(0, 16, rloop, 0)
                    return c1
                lax.fori_loop(0, 8, gloop, 0)

            def build_start(b, hb, sem):
                bv = plsc.load_gather(ebuf, [b * 128 + iota])
                s0 = jnp.sum(jnp.where(iota == 0, bv, 0))
                bstart = s0 != sent

                def bloop(g, c1):
                    grp = plsc.load_gather(ebuf, [b * 128 + g * 16 + iota])
                    plsc.store_scatter(idxbuf, [hb + g * 16 + iota],
                                       lax.bitwise_and(grp, 0xFFFF))
                    return c1
                lax.fori_loop(0, 8, bloop, 0)

                @pl.when(bstart)
                def _():
                    pltpu.async_copy(
                        h_hbm.at[idxbuf.at[pl.ds(hb, 128)]],
                        gbuf.at[pl.ds(hb, 128)], sem)
                return bstart

            def wait_g(sem):
                pltpu.make_async_copy(
                    h_hbm.at[idxbuf.at[pl.ds(0, 128)]],
                    gbuf.at[pl.ds(0, 128)], sem).wait()

            def chunk_body(ci, go):
                running = go > 0

                @pl.when(running)
                def _():
                    pltpu.sync_copy(
                        lists_ref.at[pl.ds(lbase + ci * LCH, LCH)], ebuf)

                v0 = plsc.load_gather(ebuf, [iota])
                vl = plsc.load_gather(ebuf, [LCH - 16 + iota])
                s_first = jnp.sum(jnp.where(iota == 0, v0, 0))
                s_last = jnp.sum(jnp.where(iota == 15, vl, 0))
                process = running & (s_first != sent)
                go_next = running & (s_last != sent)

                def quad(q, carry):
                    f2, f3 = carry
                    bq = q * 4
                    bs = []
                    for j in range(4):
                        bs.append(build_start(bq + j, j * 128, sems[j]))
                        lag, lagsem, lagpo = (
                            (f2, sems[2], 256) if j == 0 else
                            (f3, sems[3], 384) if j == 1 else
                            (bs[0], sems[0], 0) if j == 2 else
                            (bs[1], sems[1], 128))

                        @pl.when((lag > 0) if j < 2 else lag)
                        def _(lagsem=lagsem):
                            wait_g(lagsem)

                        @pl.when((q > 0) if j < 2 else (q >= 0))
                        def _(j=j, lagpo=lagpo):
                            apply_blk(bq + j - 2, lagpo)

                    return (lax.select(bs[2], 1, 0), lax.select(bs[3], 1, 0))

                nq = lax.select(process, LCH // 512, 0)
                f2, f3 = lax.fori_loop(0, nq, quad,
                                       (jnp.int32(0), jnp.int32(0)))

                @pl.when(f2 > 0)
                def _():
                    wait_g(sems[2])

                @pl.when(process)
                def _():
                    apply_blk(LCH // 128 - 2, 256)

                @pl.when(f3 > 0)
                def _():
                    wait_g(sems[3])

                @pl.when(process)
                def _():
                    apply_blk(LCH // 128 - 1, 384)

                return lax.select(go_next, 1, 0)

            lax.fori_loop(0, MAXCH, chunk_body, jnp.int32(1))

            pltpu.sync_copy(acc.at[pl.ds(0, SLOTS * 64)],
                            out_hbm.at[pl.ds(wid * SLOTS * 64, SLOTS * 64)])

    return list(k(*h_list, lists))


def kernel(x, edge_index, edge_type, W_ih, W_hh, b_ih, b_hh,
           weights1, bias1, weights2, bias2,
           gamma1, beta1, gamma2, beta2,
           Wself1, bself1, Wself2, bself2, osc):
    Wg = W_ih.T
    bg = (b_ih + b_hh).reshape(1, 4 * D)
    w1c = [weights1[:, :, 64 * c:64 * (c + 1)].transpose(0, 2, 1).reshape(256, 2 * D)
           for c in range(2)]
    b1 = (bself1 + 4.0 * bias1).reshape(1, 2 * D)
    w2c = [weights2[:, :, 64 * c:64 * (c + 1)].transpose(0, 2, 1).reshape(256, D)
           for c in range(4)]
    b2 = (bself2 + 4.0 * bias2).reshape(1, D)

    src2 = edge_index[0].reshape(E // 128, 128)
    dst2 = edge_index[1].reshape(E // 128, 128)
    et2 = edge_type.reshape(E // 128, 128)

    packed = _pack_stage(src2, dst2, et2).reshape(E)
    h1, h1w = _lstm_stage(x, Wg, bg)
    lists = _sc_partition(packed)

    def unpk(a):
        return a.reshape(NPAD, 256)[:N]

    a1 = unpk(_sc_agg([h1w], lists)[0])

    out1, st1 = _mix1_stage(h1, a1, Wself1.T, w1c[0], w1c[1], b1)
    h2, h2w0, h2w1 = _bnrelu_stage(out1, st1, gamma1.reshape(1, 2 * D),
                                   beta1.reshape(1, 2 * D))

    agg2 = _sc_agg([h2w0, h2w1], lists)
    a2 = [unpk(a) for a in agg2]

    out2, st2 = _mix2_stage(h2, a2[0], a2[1], Wself2.T, w2c, b2)
    return _final_stage(out2, st2, gamma2.reshape(1, D), beta2.reshape(1, D))


# speedup vs baseline: 4.7088x; 1.0443x over previous
import functools

import jax
import jax.numpy as jnp
from jax import lax
from jax.experimental import pallas as pl
from jax.experimental.pallas import tpu as pltpu
from jax.experimental.pallas import tpu_sc as plsc

N = 10000
D = 128
E = 320000
T = 4

NW = 32
NPT = 314
NPAD = NW * NPT
SLOTS = 4 * NPT
NEG = -1e30

ROWB = 2000

_PERM = [(l % 2) * 64 + l // 2 for l in range(128)]


def _bf16_bits(v):
    import struct
    u = struct.unpack('<I', struct.pack('<f', v))[0]
    upper, lower = u >> 16, u & 0xFFFF
    if lower > 0x8000 or (lower == 0x8000 and (upper & 1)):
        upper += 1
    return upper & 0xFFFF


_NEGW_U = _bf16_bits(NEG) * 0x10001
NEGW = _NEGW_U - (1 << 32) if _NEGW_U >= (1 << 31) else _NEGW_U


def _pack_rows(h):
    ra = h[:, 0:64].astype(jnp.bfloat16).astype(jnp.float32)
    rb = h[:, 64:128].astype(jnp.bfloat16).astype(jnp.float32)
    ba = lax.bitcast_convert_type(ra, jnp.int32)
    bb = lax.bitcast_convert_type(rb, jnp.int32)
    return lax.bitwise_or(lax.shift_right_logical(ba, 16),
                          lax.bitwise_and(bb, jnp.int32(-65536)))
CH = 3200
NCH = E // CH
CAP = 3360


def _lstm_body(x_ref, wg_ref, bg_ref, h1_ref, h1w_ref):
    xs = jax.nn.sigmoid(x_ref[...])
    gates = jnp.dot(xs, wg_ref[...], preferred_element_type=jnp.float32) + bg_ref[...]
    i = gates[:, 0:D]
    g = gates[:, 2 * D:3 * D]
    o = gates[:, 3 * D:4 * D]
    c = jax.nn.sigmoid(i) * jnp.tanh(g)
    h = jax.nn.sigmoid(o) * jnp.tanh(c)
    h1_ref[...] = h
    h1w_ref[...] = _pack_rows(h)


def _lstm_stage(x, Wg, bg):
    return pl.pallas_call(
        _lstm_body,
        grid=(N // ROWB,),
        in_specs=[
            pl.BlockSpec((ROWB, D), lambda m: (m, 0)),
            pl.BlockSpec((D, 4 * D), lambda m: (0, 0)),
            pl.BlockSpec((1, 4 * D), lambda m: (0, 0)),
        ],
        out_specs=[
            pl.BlockSpec((ROWB, D), lambda m: (m, 0)),
            pl.BlockSpec((ROWB, 64), lambda m: (m, 0)),
        ],
        out_shape=[
            jax.ShapeDtypeStruct((N, D), jnp.float32),
            jax.ShapeDtypeStruct((N, 64), jnp.int32),
        ],
    )(x, Wg, bg)


def _pack_body(src_ref, dst_ref, et_ref, out_ref):
    s = src_ref[...]
    d = dst_ref[...]
    t = et_ref[...]
    out_ref[...] = lax.bitwise_or(lax.shift_left(d * 4 + t, 16), s)


def _pack_stage(src2, dst2, et2):
    rows = E // 128
    return pl.pallas_call(
        _pack_body,
        grid=(1,),
        in_specs=[pl.BlockSpec((rows, 128), lambda m: (0, 0))] * 3,
        out_specs=pl.BlockSpec((rows, 128), lambda m: (0, 0)),
        out_shape=jax.ShapeDtypeStruct((rows, 128), jnp.int32),
    )(src2, dst2, et2)


def _agg_term(a_ref, wlo_ref, whi_ref):
    w = a_ref[...]
    flo = lax.bitcast_convert_type(lax.shift_left(w, 16), jnp.float32)
    fhi = lax.bitcast_convert_type(
        lax.bitwise_and(w, jnp.int32(-65536)), jnp.float32)
    flo = jnp.where(flo <= -1e29, 0.0, flo)
    fhi = jnp.where(fhi <= -1e29, 0.0, fhi)
    return (jnp.dot(flo, wlo_ref[...], preferred_element_type=jnp.float32)
            + jnp.dot(fhi, whi_ref[...], preferred_element_type=jnp.float32))


def _mix1_body(h1, a0, ws, wc0, wc1, bv, out_ref, st_ref):
    o = jnp.dot(h1[...], ws[...], preferred_element_type=jnp.float32)
    o += _agg_term(a0, wc0, wc1)
    o += bv[...]
    out_ref[...] = o
    s = jnp.concatenate([jnp.sum(o, axis=0)[None, :],
                         jnp.sum(o * o, axis=0)[None, :]], axis=0)

    @pl.when(pl.program_id(0) == 0)
    def _():
        st_ref[...] = s

    @pl.when(pl.program_id(0) != 0)
    def _():
        st_ref[...] += s


def _mix1_stage(h1, a0, ws, wc0, wc1, bv):
    return pl.pallas_call(
        _mix1_body,
        grid=(N // ROWB,),
        in_specs=[
            pl.BlockSpec((ROWB, D), lambda m: (m, 0)),
            pl.BlockSpec((ROWB, 256), lambda m: (m, 0)),
            pl.BlockSpec((D, 256), lambda m: (0, 0)),
            pl.BlockSpec((256, 256), lambda m: (0, 0)),
            pl.BlockSpec((256, 256), lambda m: (0, 0)),
            pl.BlockSpec((1, 256), lambda m: (0, 0)),
        ],
        out_specs=[
            pl.BlockSpec((ROWB, 256), lambda m: (m, 0)),
            pl.BlockSpec((2, 256), lambda m: (0, 0)),
        ],
        out_shape=[
            jax.ShapeDtypeStruct((N, 256), jnp.float32),
            jax.ShapeDtypeStruct((2, 256), jnp.float32),
        ],
    )(h1, a0, ws, wc0, wc1, bv)


def _bnrelu_body(x_ref, st_ref, g_ref, b_ref, h_ref, w0_ref, w1_ref):
    st = st_ref[...]
    mu = st[0:1, :] / N
    var = st[1:2, :] / N - mu * mu
    scale = lax.rsqrt(var + 1e-5) * g_ref[...]
    h = jnp.maximum((x_ref[...] - mu) * scale + b_ref[...], 0.0)
    h_ref[...] = h
    w0_ref[...] = _pack_rows(h[:, 0:128])
    w1_ref[...] = _pack_rows(h[:, 128:256])


def _bnrelu_stage(x, st, gamma, beta):
    return pl.pallas_call(
        _bnrelu_body,
        grid=(N // ROWB,),
        in_specs=[
            pl.BlockSpec((ROWB, 256), lambda m: (m, 0)),
            pl.BlockSpec((2, 256), lambda m: (0, 0)),
            pl.BlockSpec((1, 256), lambda m: (0, 0)),
            pl.BlockSpec((1, 256), lambda m: (0, 0)),
        ],
        out_specs=[
            pl.BlockSpec((ROWB, 256), lambda m: (m, 0)),
            pl.BlockSpec((ROWB, 64), lambda m: (m, 0)),
            pl.BlockSpec((ROWB, 64), lambda m: (m, 0)),
        ],
        out_shape=[
            jax.ShapeDtypeStruct((N, 256), jnp.float32),
            jax.ShapeDtypeStruct((N, 64), jnp.int32),
            jax.ShapeDtypeStruct((N, 64), jnp.int32),
        ],
    )(x, st, gamma, beta)


def _mix2_body(h2, a0, a1, ws, wc0, wc1, wc2, wc3, bv, out_ref, st_ref):
    o = jnp.dot(h2[...], ws[...], preferred_element_type=jnp.float32)
    o += _agg_term(a0, wc0, wc1)
    o += _agg_term(a1, wc2, wc3)
    o += bv[...]
    out_ref[...] = o
    s = jnp.concatenate([jnp.sum(o, axis=0)[None, :],
                         jnp.sum(o * o, axis=0)[None, :]], axis=0)

    @pl.when(pl.program_id(0) == 0)
    def _():
        st_ref[...] = s

    @pl.when(pl.program_id(0) != 0)
    def _():
        st_ref[...] += s


def _mix2_stage(h2, a0, a1, ws, wcs, bv):
    return pl.pallas_call(
        _mix2_body,
        grid=(N // ROWB,),
        in_specs=[
            pl.BlockSpec((ROWB, 256), lambda m: (m, 0)),
            pl.BlockSpec((ROWB, 256), lambda m: (m, 0)),
            pl.BlockSpec((ROWB, 256), lambda m: (m, 0)),
            pl.BlockSpec((256, D), lambda m: (0, 0)),
        ] + [pl.BlockSpec((256, D), lambda m: (0, 0))] * 4 + [
            pl.BlockSpec((1, D), lambda m: (0, 0)),
        ],
        out_specs=[
            pl.BlockSpec((ROWB, D), lambda m: (m, 0)),
            pl.BlockSpec((2, D), lambda m: (0, 0)),
        ],
        out_shape=[
            jax.ShapeDtypeStruct((N, D), jnp.float32),
            jax.ShapeDtypeStruct((2, D), jnp.float32),
        ],
    )(h2, a0, a1, ws, *wcs, bv)


def _final_body(x_ref, st_ref, g_ref, b_ref, out_ref):
    st = st_ref[...]
    mu = st[0:1, :] / N
    var = st[1:2, :] / N - mu * mu
    scale = lax.rsqrt(var + 1e-5) * g_ref[...]
    h = (x_ref[...] - mu) * scale + b_ref[...]
    out_ref[...] = jax.nn.sigmoid(h - 10.0)


def _final_stage(x, st, gamma, beta):
    return pl.pallas_call(
        _final_body,
        grid=(N // ROWB,),
        in_specs=[
            pl.BlockSpec((ROWB, D), lambda m: (m, 0)),
            pl.BlockSpec((2, D), lambda m: (0, 0)),
            pl.BlockSpec((1, D), lambda m: (0, 0)),
            pl.BlockSpec((1, D), lambda m: (0, 0)),
        ],
        out_specs=pl.BlockSpec((ROWB, D), lambda m: (m, 0)),
        out_shape=jax.ShapeDtypeStruct((N, D), jnp.float32),
    )(x, st, gamma, beta)


_SC_PARAMS = dict(
    compiler_params=pltpu.CompilerParams(needs_layout_passes=False,
                                         use_tc_tiling_on_sc=False),
)
LCH = 2048
ECAP = 158 * LCH
MAXCH = ECAP // LCH


def _sc_partition(packed):
    mesh = plsc.VectorSubcoreMesh(core_axis_name="c", subcore_axis_name="s")

    @functools.partial(
        pl.kernel,
        mesh=mesh,
        out_type=jax.ShapeDtypeStruct((NW * ECAP,), jnp.int32),
        scratch_types=[
            pltpu.VMEM((2 * CH,), jnp.int32),
            pltpu.VMEM((CAP,), jnp.int32),
            pltpu.SemaphoreType.DMA,
            pltpu.SemaphoreType.DMA,
        ],
        **_SC_PARAMS,
    )
    def k(packed_ref, lists, ebuf, pend, sema, semb):
        wid = lax.axis_index("s") * 2 + lax.axis_index("c")
        lo = wid * SLOTS
        hi = lo + SLOTS
        lbase = wid * ECAP
        iota = lax.iota(jnp.int32, 16)
        sentv = jnp.full((16,), lax.shift_left(hi, 16), jnp.int32)

        def initp(q, c):
            plsc.store_scatter(pend, [q * 16 + iota], sentv)
            return c
        lax.fori_loop(0, CAP // 16, initp, 0)

        def scan_flush(po, carry):
            pending, written = carry

            def step(j, pending):
                v = plsc.load_gather(ebuf, [po + j * 16 + iota])
                slot = lax.shift_right_logical(v, 16)
                mask = (slot >= lo) & (slot < hi)
                mi = mask.astype(jnp.int32)
                cs = plsc.cumsum(mi)
                cnt = jnp.sum(mi)
                pos = jnp.maximum(pending + cs - 1, 0)
                plsc.store_scatter(pend, [pos], v, mask=mask)
                return pending + cnt

            pending = lax.fori_loop(0, CH // 16, step, pending)

            nblk = lax.shift_right_logical(pending, 7)

            def wblk(b, c):
                pltpu.sync_copy(
                    pend.at[pl.ds(b * 128, 128)],
                    lists.at[pl.ds(lbase + (written + b) * 128, 128)])
                return c
            lax.fori_loop(0, nblk, wblk, 0)

            mbase = nblk * 128
            for g in range(8):
                rem = plsc.load_gather(pend, [mbase + g * 16 + iota])
                pend[g * 16:(g + 1) * 16] = rem
            return (lax.bitwise_and(pending, 127), written + nblk)

        def start_chunk(ci, half, sem):
            pltpu.async_copy(packed_ref.at[pl.ds(ci * CH, CH)],
                             ebuf.at[pl.ds(half * CH, CH)], sem)

        def wait_chunk(sem):
            pltpu.make_async_copy(packed_ref.at[pl.ds(0, CH)],
                                  ebuf.at[pl.ds(0, CH)], sem).wait()

        start_chunk(0, 0, sema)

        def pair_body(pp, carry):
            ci0 = pp * 2
            wait_chunk(sema)
            start_chunk(ci0 + 1, 1, semb)
            carry = scan_flush(0, carry)
            wait_chunk(semb)

            @pl.when(ci0 + 2 < NCH)
            def _():
                start_chunk(ci0 + 2, 0, sema)

            return scan_flush(CH, carry)

        pending, written = lax.fori_loop(0, NCH // 2, pair_body,
                                         (jnp.int32(0), jnp.int32(0)))

        def pads(q, c):
            plsc.store_scatter(pend, [pending + q * 16 + iota], sentv)
            return c
        lax.fori_loop(0, 17, pads, 0)
        ntail = lax.shift_right_logical(pending + 143, 7)

        def wtail(b, c):
            pltpu.sync_copy(
                pend.at[pl.ds(b * 128, 128)],
                lists.at[pl.ds(lbase + (written + b) * 128, 128)])
            return c
        lax.fori_loop(0, ntail, wtail, 0)
        written = written + ntail

        for q in range(8):
            pend[q * 16:(q + 1) * 16] = sentv

        def wsent(b, c):
            pltpu.sync_copy(
                pend.at[pl.ds(0, 128)],
                lists.at[pl.ds(lbase + (written + b) * 128, 128)])
            return c
        lax.fori_loop(0, LCH // 128, wsent, 0)

    return k(packed)


def _sc_agg(h_list, lists):
    nps = len(h_list)
    mesh = plsc.VectorSubcoreMesh(core_axis_name="c", subcore_axis_name="s")

    @functools.partial(
        pl.kernel,
        mesh=mesh,
        out_type=[jax.ShapeDtypeStruct((NPAD * 4 * 64,), jnp.int32)] * nps,
        scratch_types=[
            pltpu.VMEM((LCH,), jnp.int32),
            pltpu.VMEM((256,), jnp.int32),
            pltpu.VMEM((256, 64), jnp.int32),
            pltpu.VMEM(((SLOTS + 1) * 64,), jnp.int32),
            pltpu.SemaphoreType.DMA,
            pltpu.SemaphoreType.DMA,
        ],
        **_SC_PARAMS,
    )
    def k(*refs):
        h_refs = refs[:nps]
        lists_ref = refs[nps]
        out_refs = refs[nps + 1:nps + 1 + nps]
        ebuf, idxbuf, gbuf, acc, sema, semb = refs[nps + 1 + nps:]

        wid = lax.axis_index("s") * 2 + lax.axis_index("c")
        lo = wid * SLOTS
        hi = lo + SLOTS
        lbase = wid * ECAP
        iota = lax.iota(jnp.int32, 16)
        colv = [kk * 16 + iota for kk in range(4)]
        sent = lax.shift_left(hi, 16)
        negv = jnp.full((16,), NEGW, jnp.int32)

        for p in range(nps):
            h_hbm = h_refs[p]
            out_hbm = out_refs[p]

            def initb(j, c):
                plsc.store_scatter(acc, [j * 16 + iota], negv)
                return c
            lax.fori_loop(0, (SLOTS + 1) * 4, initb, 0)

            def apply_blk(b, po):
                def gloop(g, c1):
                    grp = plsc.load_gather(ebuf, [b * 128 + g * 16 + iota])
                    slotloc = lax.shift_right_logical(grp, 16) - lo
                    base = po + g * 16
                    for r in range(16):
                        rr = jnp.full((16,), r, jnp.int32)
                        sl64 = slotloc.at[rr].get(
                            mode="promise_in_bounds") * 64
                        row = jnp.full((16,), base + r, jnp.int32)
                        for kk in range(4):
                            msg = plsc.bitcast(
                                plsc.load_gather(gbuf, [row, colv[kk]]),
                                jnp.bfloat16)
                            idxk = sl64 + colv[kk]
                            cur = plsc.bitcast(
                                plsc.load_gather(acc, [idxk]), jnp.bfloat16)
                            plsc.store_scatter(
                                acc, [idxk],
                                plsc.bitcast(jnp.maximum(cur, msg),
                                             jnp.int32))
                    return c1
                lax.fori_loop(0, 8, gloop, 0)

            def build_start(b, hb, sem):
                bv = plsc.load_gather(ebuf, [b * 128 + iota])
                s0 = jnp.sum(jnp.where(iota == 0, bv, 0))
                bstart = s0 != sent

                def bloop(g, c1):
                    grp = plsc.load_gather(ebuf, [b * 128 + g * 16 + iota])
                    plsc.store_scatter(idxbuf, [hb + g * 16 + iota],
                                       lax.bitwise_and(grp, 0xFFFF))
                    return c1
                lax.fori_loop(0, 8, bloop, 0)

                @pl.when(bstart)
                def _():
                    pltpu.async_copy(
                        h_hbm.at[idxbuf.at[pl.ds(hb, 128)]],
                        gbuf.at[pl.ds(hb, 128)], sem)
                return bstart

            def wait_g(sem):
                pltpu.make_async_copy(
                    h_hbm.at[idxbuf.at[pl.ds(0, 128)]],
                    gbuf.at[pl.ds(0, 128)], sem).wait()

            def chunk_body(ci, go):
                running = go > 0

                @pl.when(running)
                def _():
                    pltpu.sync_copy(
                        lists_ref.at[pl.ds(lbase + ci * LCH, LCH)], ebuf)

                v0 = plsc.load_gather(ebuf, [iota])
                vl = plsc.load_gather(ebuf, [LCH - 16 + iota])
                s_first = jnp.sum(jnp.where(iota == 0, v0, 0))
                s_last = jnp.sum(jnp.where(iota == 15, vl, 0))
                process = running & (s_first != sent)
                go_next = running & (s_last != sent)

                def pair(pp, started_odd):
                    b0 = pp * 2
                    bs0 = build_start(b0, 0, sema)

                    @pl.when(started_odd > 0)
                    def _():
                        wait_g(semb)

                    @pl.when(pp > 0)
                    def _():
                        apply_blk(b0 - 1, 128)

                    bs1 = build_start(b0 + 1, 128, semb)

                    @pl.when(bs0)
                    def _():
                        wait_g(sema)

                    apply_blk(b0, 0)
                    return lax.select(bs1, 1, 0)

                npair = lax.select(process, LCH // 256, 0)
                started_odd = lax.fori_loop(0, npair, pair, jnp.int32(0))

                @pl.when(started_odd > 0)
                def _():
                    wait_g(semb)

                @pl.when(process)
                def _():
                    apply_blk(LCH // 128 - 1, 128)

                return lax.select(go_next, 1, 0)

            lax.fori_loop(0, MAXCH, chunk_body, jnp.int32(1))

            pltpu.sync_copy(acc.at[pl.ds(0, SLOTS * 64)],
                            out_hbm.at[pl.ds(wid * SLOTS * 64, SLOTS * 64)])

    return list(k(*h_list, lists))


def kernel(x, edge_index, edge_type, W_ih, W_hh, b_ih, b_hh,
           weights1, bias1, weights2, bias2,
           gamma1, beta1, gamma2, beta2,
           Wself1, bself1, Wself2, bself2, osc):
    Wg = W_ih.T
    bg = (b_ih + b_hh).reshape(1, 4 * D)
    w1c = [weights1[:, :, 64 * c:64 * (c + 1)].transpose(0, 2, 1).reshape(256, 2 * D)
           for c in range(2)]
    b1 = (bself1 + 4.0 * bias1).reshape(1, 2 * D)
    w2c = [weights2[:, :, 64 * c:64 * (c + 1)].transpose(0, 2, 1).reshape(256, D)
           for c in range(4)]
    b2 = (bself2 + 4.0 * bias2).reshape(1, D)

    src2 = edge_index[0].reshape(E // 128, 128)
    dst2 = edge_index[1].reshape(E // 128, 128)
    et2 = edge_type.reshape(E // 128, 128)

    packed = _pack_stage(src2, dst2, et2).reshape(E)
    h1, h1w = _lstm_stage(x, Wg, bg)
    lists = _sc_partition(packed)

    def unpk(a):
        return a.reshape(NPAD, 256)[:N]

    a1 = unpk(_sc_agg([h1w], lists)[0])

    out1, st1 = _mix1_stage(h1, a1, Wself1.T, w1c[0], w1c[1], b1)
    h2, h2w0, h2w1 = _bnrelu_stage(out1, st1, gamma1.reshape(1, 2 * D),
                                   beta1.reshape(1, 2 * D))

    agg2 = _sc_agg([h2w0, h2w1], lists)
    a2 = [unpk(a) for a in agg2]

    out2, st2 = _mix2_stage(h2, a2[0], a2[1], Wself2.T, w2c, b2)
    return _final_stage(out2, st2, gamma2.reshape(1, D), beta2.reshape(1, D))


# speedup vs baseline: 5.1320x; 1.0899x over previous
import functools

import jax
import jax.numpy as jnp
from jax import lax
from jax.experimental import pallas as pl
from jax.experimental.pallas import tpu as pltpu
from jax.experimental.pallas import tpu_sc as plsc

N = 10000
D = 128
E = 320000
T = 4

NW = 32
NPT = 314
NPAD = NW * NPT
SLOTS = 4 * NPT
NEG = -1e30

ROWB = 2000

_PERM = [(l % 2) * 64 + l // 2 for l in range(128)]


def _bf16_bits(v):
    import struct
    u = struct.unpack('<I', struct.pack('<f', v))[0]
    upper, lower = u >> 16, u & 0xFFFF
    if lower > 0x8000 or (lower == 0x8000 and (upper & 1)):
        upper += 1
    return upper & 0xFFFF


_NEGW_U = _bf16_bits(NEG) * 0x10001
NEGW = _NEGW_U - (1 << 32) if _NEGW_U >= (1 << 31) else _NEGW_U


def _pack_rows(h):
    ra = h[:, 0:64].astype(jnp.bfloat16).astype(jnp.float32)
    rb = h[:, 64:128].astype(jnp.bfloat16).astype(jnp.float32)
    ba = lax.bitcast_convert_type(ra, jnp.int32)
    bb = lax.bitcast_convert_type(rb, jnp.int32)
    return lax.bitwise_or(lax.shift_right_logical(ba, 16),
                          lax.bitwise_and(bb, jnp.int32(-65536)))
CH = 3200
NCH = E // CH
CAP = 3360


def _lstm_body(x_ref, wg_ref, bg_ref, h1_ref, h1w_ref):
    xs = jax.nn.sigmoid(x_ref[...])
    gates = jnp.dot(xs, wg_ref[...], preferred_element_type=jnp.float32) + bg_ref[...]
    i = gates[:, 0:D]
    g = gates[:, 2 * D:3 * D]
    o = gates[:, 3 * D:4 * D]
    c = jax.nn.sigmoid(i) * jnp.tanh(g)
    h = jax.nn.sigmoid(o) * jnp.tanh(c)
    h1_ref[...] = h
    h1w_ref[...] = _pack_rows(h)


def _lstm_stage(x, Wg, bg):
    return pl.pallas_call(
        _lstm_body,
        grid=(N // ROWB,),
        in_specs=[
            pl.BlockSpec((ROWB, D), lambda m: (m, 0)),
            pl.BlockSpec((D, 4 * D), lambda m: (0, 0)),
            pl.BlockSpec((1, 4 * D), lambda m: (0, 0)),
        ],
        out_specs=[
            pl.BlockSpec((ROWB, D), lambda m: (m, 0)),
            pl.BlockSpec((ROWB, 64), lambda m: (m, 0)),
        ],
        out_shape=[
            jax.ShapeDtypeStruct((N, D), jnp.float32),
            jax.ShapeDtypeStruct((N, 64), jnp.int32),
        ],
    )(x, Wg, bg)


def _pack_body(src_ref, dst_ref, et_ref, out_ref):
    s = src_ref[...]
    d = dst_ref[...]
    t = et_ref[...]
    out_ref[...] = lax.bitwise_or(lax.shift_left(d * 4 + t, 16), s)


def _pack_stage(src2, dst2, et2):
    rows = E // 128
    return pl.pallas_call(
        _pack_body,
        grid=(1,),
        in_specs=[pl.BlockSpec((rows, 128), lambda m: (0, 0))] * 3,
        out_specs=pl.BlockSpec((rows, 128), lambda m: (0, 0)),
        out_shape=jax.ShapeDtypeStruct((rows, 128), jnp.int32),
    )(src2, dst2, et2)


def _agg_term(a_ref, wlo_ref, whi_ref):
    w = a_ref[...]
    flo = lax.bitcast_convert_type(lax.shift_left(w, 16), jnp.float32)
    fhi = lax.bitcast_convert_type(
        lax.bitwise_and(w, jnp.int32(-65536)), jnp.float32)
    flo = jnp.where(flo <= -1e29, 0.0, flo)
    fhi = jnp.where(fhi <= -1e29, 0.0, fhi)
    return (jnp.dot(flo, wlo_ref[...], preferred_element_type=jnp.float32)
            + jnp.dot(fhi, whi_ref[...], preferred_element_type=jnp.float32))


def _mix1_body(h1, a0, ws, wc0, wc1, bv, out_ref, st_ref):
    o = jnp.dot(h1[...], ws[...], preferred_element_type=jnp.float32)
    o += _agg_term(a0, wc0, wc1)
    o += bv[...]
    out_ref[...] = o
    s = jnp.concatenate([jnp.sum(o, axis=0)[None, :],
                         jnp.sum(o * o, axis=0)[None, :]], axis=0)

    @pl.when(pl.program_id(0) == 0)
    def _():
        st_ref[...] = s

    @pl.when(pl.program_id(0) != 0)
    def _():
        st_ref[...] += s


def _mix1_stage(h1, a0, ws, wc0, wc1, bv):
    return pl.pallas_call(
        _mix1_body,
        grid=(N // ROWB,),
        in_specs=[
            pl.BlockSpec((ROWB, D), lambda m: (m, 0)),
            pl.BlockSpec((ROWB, 256), lambda m: (m, 0)),
            pl.BlockSpec((D, 256), lambda m: (0, 0)),
            pl.BlockSpec((256, 256), lambda m: (0, 0)),
            pl.BlockSpec((256, 256), lambda m: (0, 0)),
            pl.BlockSpec((1, 256), lambda m: (0, 0)),
        ],
        out_specs=[
            pl.BlockSpec((ROWB, 256), lambda m: (m, 0)),
            pl.BlockSpec((2, 256), lambda m: (0, 0)),
        ],
        out_shape=[
            jax.ShapeDtypeStruct((N, 256), jnp.float32),
            jax.ShapeDtypeStruct((2, 256), jnp.float32),
        ],
    )(h1, a0, ws, wc0, wc1, bv)


def _bnrelu_body(x_ref, st_ref, g_ref, b_ref, h_ref, w0_ref, w1_ref):
    st = st_ref[...]
    mu = st[0:1, :] / N
    var = st[1:2, :] / N - mu * mu
    scale = lax.rsqrt(var + 1e-5) * g_ref[...]
    h = jnp.maximum((x_ref[...] - mu) * scale + b_ref[...], 0.0)
    h_ref[...] = h
    w0_ref[...] = _pack_rows(h[:, 0:128])
    w1_ref[...] = _pack_rows(h[:, 128:256])


def _bnrelu_stage(x, st, gamma, beta):
    return pl.pallas_call(
        _bnrelu_body,
        grid=(N // ROWB,),
        in_specs=[
            pl.BlockSpec((ROWB, 256), lambda m: (m, 0)),
            pl.BlockSpec((2, 256), lambda m: (0, 0)),
            pl.BlockSpec((1, 256), lambda m: (0, 0)),
            pl.BlockSpec((1, 256), lambda m: (0, 0)),
        ],
        out_specs=[
            pl.BlockSpec((ROWB, 256), lambda m: (m, 0)),
            pl.BlockSpec((ROWB, 64), lambda m: (m, 0)),
            pl.BlockSpec((ROWB, 64), lambda m: (m, 0)),
        ],
        out_shape=[
            jax.ShapeDtypeStruct((N, 256), jnp.float32),
            jax.ShapeDtypeStruct((N, 64), jnp.int32),
            jax.ShapeDtypeStruct((N, 64), jnp.int32),
        ],
    )(x, st, gamma, beta)


def _mix2_body(h2, a0, a1, ws, wc0, wc1, wc2, wc3, bv, out_ref, st_ref):
    o = jnp.dot(h2[...], ws[...], preferred_element_type=jnp.float32)
    o += _agg_term(a0, wc0, wc1)
    o += _agg_term(a1, wc2, wc3)
    o += bv[...]
    out_ref[...] = o
    s = jnp.concatenate([jnp.sum(o, axis=0)[None, :],
                         jnp.sum(o * o, axis=0)[None, :]], axis=0)

    @pl.when(pl.program_id(0) == 0)
    def _():
        st_ref[...] = s

    @pl.when(pl.program_id(0) != 0)
    def _():
        st_ref[...] += s


def _mix2_stage(h2, a0, a1, ws, wcs, bv):
    return pl.pallas_call(
        _mix2_body,
        grid=(N // ROWB,),
        in_specs=[
            pl.BlockSpec((ROWB, 256), lambda m: (m, 0)),
            pl.BlockSpec((ROWB, 256), lambda m: (m, 0)),
            pl.BlockSpec((ROWB, 256), lambda m: (m, 0)),
            pl.BlockSpec((256, D), lambda m: (0, 0)),
        ] + [pl.BlockSpec((256, D), lambda m: (0, 0))] * 4 + [
            pl.BlockSpec((1, D), lambda m: (0, 0)),
        ],
        out_specs=[
            pl.BlockSpec((ROWB, D), lambda m: (m, 0)),
            pl.BlockSpec((2, D), lambda m: (0, 0)),
        ],
        out_shape=[
            jax.ShapeDtypeStruct((N, D), jnp.float32),
            jax.ShapeDtypeStruct((2, D), jnp.float32),
        ],
    )(h2, a0, a1, ws, *wcs, bv)


def _final_body(x_ref, st_ref, g_ref, b_ref, out_ref):
    st = st_ref[...]
    mu = st[0:1, :] / N
    var = st[1:2, :] / N - mu * mu
    scale = lax.rsqrt(var + 1e-5) * g_ref[...]
    h = (x_ref[...] - mu) * scale + b_ref[...]
    out_ref[...] = jax.nn.sigmoid(h - 10.0)


def _final_stage(x, st, gamma, beta):
    return pl.pallas_call(
        _final_body,
        grid=(N // ROWB,),
        in_specs=[
            pl.BlockSpec((ROWB, D), lambda m: (m, 0)),
            pl.BlockSpec((2, D), lambda m: (0, 0)),
            pl.BlockSpec((1, D), lambda m: (0, 0)),
            pl.BlockSpec((1, D), lambda m: (0, 0)),
        ],
        out_specs=pl.BlockSpec((ROWB, D), lambda m: (m, 0)),
        out_shape=jax.ShapeDtypeStruct((N, D), jnp.float32),
    )(x, st, gamma, beta)


_SC_PARAMS = dict(
    compiler_params=pltpu.CompilerParams(needs_layout_passes=False,
                                         use_tc_tiling_on_sc=False),
)
LCH = 2048
ECAP = 158 * LCH
MAXCH = ECAP // LCH


def _sc_partition(packed):
    mesh = plsc.VectorSubcoreMesh(core_axis_name="c", subcore_axis_name="s")

    @functools.partial(
        pl.kernel,
        mesh=mesh,
        out_type=jax.ShapeDtypeStruct((NW * ECAP,), jnp.int32),
        scratch_types=[
            pltpu.VMEM((2 * CH,), jnp.int32),
            pltpu.VMEM((CAP,), jnp.int32),
            pltpu.SemaphoreType.DMA,
            pltpu.SemaphoreType.DMA,
        ],
        **_SC_PARAMS,
    )
    def k(packed_ref, lists, ebuf, pend, sema, semb):
        wid = lax.axis_index("s") * 2 + lax.axis_index("c")
        lo = wid * SLOTS
        hi = lo + SLOTS
        lbase = wid * ECAP
        iota = lax.iota(jnp.int32, 16)
        sentv = jnp.full((16,), lax.shift_left(hi, 16), jnp.int32)

        def initp(q, c):
            plsc.store_scatter(pend, [q * 16 + iota], sentv)
            return c
        lax.fori_loop(0, CAP // 16, initp, 0)

        def scan_flush(po, carry):
            pending, written = carry

            def step(j, pending):
                v = plsc.load_gather(ebuf, [po + j * 16 + iota])
                slot = lax.shift_right_logical(v, 16)
                mask = (slot >= lo) & (slot < hi)
                mi = mask.astype(jnp.int32)
                cs = plsc.cumsum(mi)
                cnt = jnp.sum(mi)
                pos = jnp.maximum(pending + cs - 1, 0)
                plsc.store_scatter(pend, [pos], v, mask=mask)
                return pending + cnt

            pending = lax.fori_loop(0, CH // 16, step, pending)

            nblk = lax.shift_right_logical(pending, 7)

            def wblk(b, c):
                pltpu.sync_copy(
                    pend.at[pl.ds(b * 128, 128)],
                    lists.at[pl.ds(lbase + (written + b) * 128, 128)])
                return c
            lax.fori_loop(0, nblk, wblk, 0)

            mbase = nblk * 128
            for g in range(8):
                rem = plsc.load_gather(pend, [mbase + g * 16 + iota])
                pend[g * 16:(g + 1) * 16] = rem
            return (lax.bitwise_and(pending, 127), written + nblk)

        def start_chunk(ci, half, sem):
            pltpu.async_copy(packed_ref.at[pl.ds(ci * CH, CH)],
                             ebuf.at[pl.ds(half * CH, CH)], sem)

        def wait_chunk(sem):
            pltpu.make_async_copy(packed_ref.at[pl.ds(0, CH)],
                                  ebuf.at[pl.ds(0, CH)], sem).wait()

        start_chunk(0, 0, sema)

        def pair_body(pp, carry):
            ci0 = pp * 2
            wait_chunk(sema)
            start_chunk(ci0 + 1, 1, semb)
            carry = scan_flush(0, carry)
            wait_chunk(semb)

            @pl.when(ci0 + 2 < NCH)
            def _():
                start_chunk(ci0 + 2, 0, sema)

            return scan_flush(CH, carry)

        pending, written = lax.fori_loop(0, NCH // 2, pair_body,
                                         (jnp.int32(0), jnp.int32(0)))

        def pads(q, c):
            plsc.store_scatter(pend, [pending + q * 16 + iota], sentv)
            return c
        lax.fori_loop(0, 17, pads, 0)
        ntail = lax.shift_right_logical(pending + 143, 7)

        def wtail(b, c):
            pltpu.sync_copy(
                pend.at[pl.ds(b * 128, 128)],
                lists.at[pl.ds(lbase + (written + b) * 128, 128)])
            return c
        lax.fori_loop(0, ntail, wtail, 0)
        written = written + ntail

        for q in range(8):
            pend[q * 16:(q + 1) * 16] = sentv

        def wsent(b, c):
            pltpu.sync_copy(
                pend.at[pl.ds(0, 128)],
                lists.at[pl.ds(lbase + (written + b) * 128, 128)])
            return c
        lax.fori_loop(0, LCH // 128, wsent, 0)

    return k(packed)


def _sc_agg(h_list, lists):
    nps = len(h_list)
    mesh = plsc.VectorSubcoreMesh(core_axis_name="c", subcore_axis_name="s")

    @functools.partial(
        pl.kernel,
        mesh=mesh,
        out_type=[jax.ShapeDtypeStruct((NPAD * 4 * 64,), jnp.int32)] * nps,
        scratch_types=[
            pltpu.VMEM((LCH,), jnp.int32),
            pltpu.VMEM((256,), jnp.int32),
            pltpu.VMEM((256, 64), jnp.int32),
            pltpu.VMEM(((SLOTS + 1) * 64,), jnp.int32),
            pltpu.SemaphoreType.DMA,
            pltpu.SemaphoreType.DMA,
        ],
        **_SC_PARAMS,
    )
    def k(*refs):
        h_refs = refs[:nps]
        lists_ref = refs[nps]
        out_refs = refs[nps + 1:nps + 1 + nps]
        ebuf, idxbuf, gbuf, acc, sema, semb = refs[nps + 1 + nps:]

        wid = lax.axis_index("s") * 2 + lax.axis_index("c")
        lo = wid * SLOTS
        hi = lo + SLOTS
        lbase = wid * ECAP
        iota = lax.iota(jnp.int32, 16)
        colv = [kk * 16 + iota for kk in range(4)]
        sent = lax.shift_left(hi, 16)
        negv = jnp.full((16,), NEGW, jnp.int32)

        for p in range(nps):
            h_hbm = h_refs[p]
            out_hbm = out_refs[p]

            def initb(j, c):
                plsc.store_scatter(acc, [j * 16 + iota], negv)
                return c
            lax.fori_loop(0, (SLOTS + 1) * 4, initb, 0)

            def apply_blk(b, po):
                def gloop(g, c1):
                    grp = plsc.load_gather(ebuf, [b * 128 + g * 16 + iota])
                    slotloc = lax.shift_right_logical(grp, 16) - lo
                    base = po + g * 16
                    for r in range(16):
                        rr = jnp.full((16,), r, jnp.int32)
                        sl64 = slotloc.at[rr].get(
                            mode="promise_in_bounds") * 64
                        row = jnp.full((16,), base + r, jnp.int32)
                        for kk in range(4):
                            msg = plsc.bitcast(
                                plsc.load_gather(gbuf, [row, colv[kk]]),
                                jnp.bfloat16)
                            idxk = sl64 + colv[kk]
                            cur = plsc.bitcast(
                                plsc.load_gather(acc, [idxk]), jnp.bfloat16)
                            plsc.store_scatter(
                                acc, [idxk],
                                plsc.bitcast(jnp.maximum(cur, msg),
                                             jnp.int32))
                    return c1
                lax.fori_loop(0, 8, gloop, 0)

            def build_start(b, hb, sem):
                bv = plsc.load_gather(ebuf, [b * 128 + iota])
                s0 = jnp.sum(jnp.where(iota == 0, bv, 0))
                bstart = s0 != sent

                def bloop(g, c1):
                    grp = plsc.load_gather(ebuf, [b * 128 + g * 16 + iota])
                    plsc.store_scatter(idxbuf, [hb + g * 16 + iota],
                                       lax.bitwise_and(grp, 0xFFFF))
                    return c1
                lax.fori_loop(0, 8, bloop, 0)

                @pl.when(bstart)
                def _():
                    pltpu.async_copy(
                        h_hbm.at[idxbuf.at[pl.ds(hb, 128)]],
                        gbuf.at[pl.ds(hb, 128)], sem)
                return bstart

            def wait_g(sem):
                pltpu.make_async_copy(
                    h_hbm.at[idxbuf.at[pl.ds(0, 128)]],
                    gbuf.at[pl.ds(0, 128)], sem).wait()

            def chunk_body(ci, go):
                running = go > 0

                @pl.when(running)
                def _():
                    pltpu.sync_copy(
                        lists_ref.at[pl.ds(lbase + ci * LCH, LCH)], ebuf)

                v0 = plsc.load_gather(ebuf, [iota])
                vl = plsc.load_gather(ebuf, [LCH - 16 + iota])
                s_first = jnp.sum(jnp.where(iota == 0, v0, 0))
                s_last = jnp.sum(jnp.where(iota == 15, vl, 0))
                process = running & (s_first != sent)
                go_next = running & (s_last != sent)

                def pair(pp, started_odd):
                    b0 = pp * 2
                    bs0 = build_start(b0, 0, sema)

                    @pl.when(started_odd > 0)
                    def _():
                        wait_g(semb)
                        apply_blk(b0 - 1, 128)

                    bs1 = build_start(b0 + 1, 128, semb)

                    @pl.when(bs0)
                    def _():
                        wait_g(sema)
                        apply_blk(b0, 0)

                    return lax.select(bs1, 1, 0)

                npair = lax.select(process, LCH // 256, 0)
                started_odd = lax.fori_loop(0, npair, pair, jnp.int32(0))

                @pl.when(started_odd > 0)
                def _():
                    wait_g(semb)
                    apply_blk(LCH // 128 - 1, 128)

                return lax.select(go_next, 1, 0)

            lax.fori_loop(0, MAXCH, chunk_body, jnp.int32(1))

            pltpu.sync_copy(acc.at[pl.ds(0, SLOTS * 64)],
                            out_hbm.at[pl.ds(wid * SLOTS * 64, SLOTS * 64)])

    return list(k(*h_list, lists))


def kernel(x, edge_index, edge_type, W_ih, W_hh, b_ih, b_hh,
           weights1, bias1, weights2, bias2,
           gamma1, beta1, gamma2, beta2,
           Wself1, bself1, Wself2, bself2, osc):
    Wg = W_ih.T
    bg = (b_ih + b_hh).reshape(1, 4 * D)
    w1c = [weights1[:, :, 64 * c:64 * (c + 1)].transpose(0, 2, 1).reshape(256, 2 * D)
           for c in range(2)]
    b1 = (bself1 + 4.0 * bias1).reshape(1, 2 * D)
    w2c = [weights2[:, :, 64 * c:64 * (c + 1)].transpose(0, 2, 1).reshape(256, D)
           for c in range(4)]
    b2 = (bself2 + 4.0 * bias2).reshape(1, D)

    src2 = edge_index[0].reshape(E // 128, 128)
    dst2 = edge_index[1].reshape(E // 128, 128)
    et2 = edge_type.reshape(E // 128, 128)

    packed = _pack_stage(src2, dst2, et2).reshape(E)
    h1, h1w = _lstm_stage(x, Wg, bg)
    lists = _sc_partition(packed)

    def unpk(a):
        return a.reshape(NPAD, 256)[:N]

    a1 = unpk(_sc_agg([h1w], lists)[0])

    out1, st1 = _mix1_stage(h1, a1, Wself1.T, w1c[0], w1c[1], b1)
    h2, h2w0, h2w1 = _bnrelu_stage(out1, st1, gamma1.reshape(1, 2 * D),
                                   beta1.reshape(1, 2 * D))

    agg2 = _sc_agg([h2w0, h2w1], lists)
    a2 = [unpk(a) for a in agg2]

    out2, st2 = _mix2_stage(h2, a2[0], a2[1], Wself2.T, w2c, b2)
    return _final_stage(out2, st2, gamma2.reshape(1, D), beta2.reshape(1, D))


# speedup vs baseline: 5.1320x; 1.0000x over previous
import functools

import jax
import jax.numpy as jnp
from jax import lax
from jax.experimental import pallas as pl
from jax.experimental.pallas import tpu as pltpu
from jax.experimental.pallas import tpu_sc as plsc

N = 10000
D = 128
E = 320000
T = 4

NW = 32
NPT = 314
NPAD = NW * NPT
SLOTS = 4 * NPT
NEG = -1e30

ROWB = 2000


def _bf16_bits(v):
    import struct
    u = struct.unpack('<I', struct.pack('<f', v))[0]
    upper, lower = u >> 16, u & 0xFFFF
    if lower > 0x8000 or (lower == 0x8000 and (upper & 1)):
        upper += 1
    return upper & 0xFFFF


_NEGW_U = _bf16_bits(NEG) * 0x10001
NEGW = _NEGW_U - (1 << 32) if _NEGW_U >= (1 << 31) else _NEGW_U


def _pack_rows(h):
    ra = h[:, 0:64].astype(jnp.bfloat16).astype(jnp.float32)
    rb = h[:, 64:128].astype(jnp.bfloat16).astype(jnp.float32)
    ba = lax.bitcast_convert_type(ra, jnp.int32)
    bb = lax.bitcast_convert_type(rb, jnp.int32)
    return lax.bitwise_or(lax.shift_right_logical(ba, 16),
                          lax.bitwise_and(bb, jnp.int32(-65536)))
CH = 3200
NCH = E // CH
CAP = 3360


def _lstm_body(x_ref, wg_ref, bg_ref, h1_ref, h1w_ref):
    xs = jax.nn.sigmoid(x_ref[...])
    gates = jnp.dot(xs, wg_ref[...], preferred_element_type=jnp.float32) + bg_ref[...]
    i = gates[:, 0:D]
    g = gates[:, 2 * D:3 * D]
    o = gates[:, 3 * D:4 * D]
    c = jax.nn.sigmoid(i) * jnp.tanh(g)
    h = jax.nn.sigmoid(o) * jnp.tanh(c)
    h1_ref[...] = h
    h1w_ref[...] = _pack_rows(h)


def _lstm_stage(x, Wg, bg):
    return pl.pallas_call(
        _lstm_body,
        grid=(N // ROWB,),
        in_specs=[
            pl.BlockSpec((ROWB, D), lambda m: (m, 0)),
            pl.BlockSpec((D, 4 * D), lambda m: (0, 0)),
            pl.BlockSpec((1, 4 * D), lambda m: (0, 0)),
        ],
        out_specs=[
            pl.BlockSpec((ROWB, D), lambda m: (m, 0)),
            pl.BlockSpec((ROWB, 64), lambda m: (m, 0)),
        ],
        out_shape=[
            jax.ShapeDtypeStruct((N, D), jnp.float32),
            jax.ShapeDtypeStruct((N, 64), jnp.int32),
        ],
    )(x, Wg, bg)


def _pack_body(src_ref, dst_ref, et_ref, out_ref):
    s = src_ref[...]
    d = dst_ref[...]
    t = et_ref[...]
    out_ref[...] = lax.bitwise_or(lax.shift_left(d * 4 + t, 16), s)


def _pack_stage(src2, dst2, et2):
    rows = E // 128
    return pl.pallas_call(
        _pack_body,
        grid=(1,),
        in_specs=[pl.BlockSpec((rows, 128), lambda m: (0, 0))] * 3,
        out_specs=pl.BlockSpec((rows, 128), lambda m: (0, 0)),
        out_shape=jax.ShapeDtypeStruct((rows, 128), jnp.int32),
    )(src2, dst2, et2)


def _agg_term(a_ref, wlo_ref, whi_ref):
    w = a_ref[...]
    flo = lax.bitcast_convert_type(lax.shift_left(w, 16), jnp.float32)
    fhi = lax.bitcast_convert_type(
        lax.bitwise_and(w, jnp.int32(-65536)), jnp.float32)
    flo = jnp.where(flo <= -1e29, 0.0, flo)
    fhi = jnp.where(fhi <= -1e29, 0.0, fhi)
    return (jnp.dot(flo, wlo_ref[...], preferred_element_type=jnp.float32)
            + jnp.dot(fhi, whi_ref[...], preferred_element_type=jnp.float32))


def _mix1_body(h1, a0, ws, wc0, wc1, bv, out_ref, st_ref):
    o = jnp.dot(h1[...], ws[...], preferred_element_type=jnp.float32)
    o += _agg_term(a0, wc0, wc1)
    o += bv[...]
    out_ref[...] = o
    s = jnp.concatenate([jnp.sum(o, axis=0)[None, :],
                         jnp.sum(o * o, axis=0)[None, :]], axis=0)

    @pl.when(pl.program_id(0) == 0)
    def _():
        st_ref[...] = s

    @pl.when(pl.program_id(0) != 0)
    def _():
        st_ref[...] += s


def _mix1_stage(h1, a0, ws, wc0, wc1, bv):
    return pl.pallas_call(
        _mix1_body,
        grid=(N // ROWB,),
        in_specs=[
            pl.BlockSpec((ROWB, D), lambda m: (m, 0)),
            pl.BlockSpec((ROWB, 256), lambda m: (m, 0)),
            pl.BlockSpec((D, 256), lambda m: (0, 0)),
            pl.BlockSpec((256, 256), lambda m: (0, 0)),
            pl.BlockSpec((256, 256), lambda m: (0, 0)),
            pl.BlockSpec((1, 256), lambda m: (0, 0)),
        ],
        out_specs=[
            pl.BlockSpec((ROWB, 256), lambda m: (m, 0)),
            pl.BlockSpec((2, 256), lambda m: (0, 0)),
        ],
        out_shape=[
            jax.ShapeDtypeStruct((N, 256), jnp.float32),
            jax.ShapeDtypeStruct((2, 256), jnp.float32),
        ],
    )(h1, a0, ws, wc0, wc1, bv)


def _bnrelu_body(x_ref, st_ref, g_ref, b_ref, h_ref, w0_ref, w1_ref):
    st = st_ref[...]
    mu = st[0:1, :] / N
    var = st[1:2, :] / N - mu * mu
    scale = lax.rsqrt(var + 1e-5) * g_ref[...]
    h = jnp.maximum((x_ref[...] - mu) * scale + b_ref[...], 0.0)
    h_ref[...] = h
    w0_ref[...] = _pack_rows(h[:, 0:128])
    w1_ref[...] = _pack_rows(h[:, 128:256])


def _bnrelu_stage(x, st, gamma, beta):
    return pl.pallas_call(
        _bnrelu_body,
        grid=(N // ROWB,),
        in_specs=[
            pl.BlockSpec((ROWB, 256), lambda m: (m, 0)),
            pl.BlockSpec((2, 256), lambda m: (0, 0)),
            pl.BlockSpec((1, 256), lambda m: (0, 0)),
            pl.BlockSpec((1, 256), lambda m: (0, 0)),
        ],
        out_specs=[
            pl.BlockSpec((ROWB, 256), lambda m: (m, 0)),
            pl.BlockSpec((ROWB, 64), lambda m: (m, 0)),
            pl.BlockSpec((ROWB, 64), lambda m: (m, 0)),
        ],
        out_shape=[
            jax.ShapeDtypeStruct((N, 256), jnp.float32),
            jax.ShapeDtypeStruct((N, 64), jnp.int32),
            jax.ShapeDtypeStruct((N, 64), jnp.int32),
        ],
    )(x, st, gamma, beta)


def _mix2_body(h2, a0, a1, ws, wc0, wc1, wc2, wc3, bv, out_ref, st_ref):
    o = jnp.dot(h2[...], ws[...], preferred_element_type=jnp.float32)
    o += _agg_term(a0, wc0, wc1)
    o += _agg_term(a1, wc2, wc3)
    o += bv[...]
    out_ref[...] = o
    s = jnp.concatenate([jnp.sum(o, axis=0)[None, :],
                         jnp.sum(o * o, axis=0)[None, :]], axis=0)

    @pl.when(pl.program_id(0) == 0)
    def _():
        st_ref[...] = s

    @pl.when(pl.program_id(0) != 0)
    def _():
        st_ref[...] += s


def _mix2_stage(h2, a0, a1, ws, wcs, bv):
    return pl.pallas_call(
        _mix2_body,
        grid=(N // ROWB,),
        in_specs=[
            pl.BlockSpec((ROWB, 256), lambda m: (m, 0)),
            pl.BlockSpec((ROWB, 256), lambda m: (m, 0)),
            pl.BlockSpec((ROWB, 256), lambda m: (m, 0)),
            pl.BlockSpec((256, D), lambda m: (0, 0)),
        ] + [pl.BlockSpec((256, D), lambda m: (0, 0))] * 4 + [
            pl.BlockSpec((1, D), lambda m: (0, 0)),
        ],
        out_specs=[
            pl.BlockSpec((ROWB, D), lambda m: (m, 0)),
            pl.BlockSpec((2, D), lambda m: (0, 0)),
        ],
        out_shape=[
            jax.ShapeDtypeStruct((N, D), jnp.float32),
            jax.ShapeDtypeStruct((2, D), jnp.float32),
        ],
    )(h2, a0, a1, ws, *wcs, bv)


def _final_body(x_ref, st_ref, g_ref, b_ref, out_ref):
    st = st_ref[...]
    mu = st[0:1, :] / N
    var = st[1:2, :] / N - mu * mu
    scale = lax.rsqrt(var + 1e-5) * g_ref[...]
    h = (x_ref[...] - mu) * scale + b_ref[...]
    out_ref[...] = jax.nn.sigmoid(h - 10.0)


def _final_stage(x, st, gamma, beta):
    return pl.pallas_call(
        _final_body,
        grid=(N // ROWB,),
        in_specs=[
            pl.BlockSpec((ROWB, D), lambda m: (m, 0)),
            pl.BlockSpec((2, D), lambda m: (0, 0)),
            pl.BlockSpec((1, D), lambda m: (0, 0)),
            pl.BlockSpec((1, D), lambda m: (0, 0)),
        ],
        out_specs=pl.BlockSpec((ROWB, D), lambda m: (m, 0)),
        out_shape=jax.ShapeDtypeStruct((N, D), jnp.float32),
    )(x, st, gamma, beta)


_SC_PARAMS = dict(
    compiler_params=pltpu.CompilerParams(needs_layout_passes=False,
                                         use_tc_tiling_on_sc=False),
)
LCH = 2048
ECAP = 158 * LCH
MAXCH = ECAP // LCH


def _sc_partition(packed):
    mesh = plsc.VectorSubcoreMesh(core_axis_name="c", subcore_axis_name="s")

    @functools.partial(
        pl.kernel,
        mesh=mesh,
        out_type=jax.ShapeDtypeStruct((NW * ECAP,), jnp.int32),
        scratch_types=[
            pltpu.VMEM((2 * CH,), jnp.int32),
            pltpu.VMEM((CAP,), jnp.int32),
            pltpu.SemaphoreType.DMA,
            pltpu.SemaphoreType.DMA,
        ],
        **_SC_PARAMS,
    )
    def k(packed_ref, lists, ebuf, pend, sema, semb):
        wid = lax.axis_index("s") * 2 + lax.axis_index("c")
        lo = wid * SLOTS
        hi = lo + SLOTS
        lbase = wid * ECAP
        iota = lax.iota(jnp.int32, 16)
        sentv = jnp.full((16,), lax.shift_left(hi, 16), jnp.int32)

        def initp(q, c):
            plsc.store_scatter(pend, [q * 16 + iota], sentv)
            return c
        lax.fori_loop(0, CAP // 16, initp, 0)

        def scan_flush(po, carry):
            pending, written = carry

            def step(j, pending):
                v = plsc.load_gather(ebuf, [po + j * 16 + iota])
                slot = lax.shift_right_logical(v, 16)
                mask = (slot >= lo) & (slot < hi)
                mi = mask.astype(jnp.int32)
                cs = plsc.cumsum(mi)
                cnt = jnp.sum(mi)
                pos = jnp.maximum(pending + cs - 1, 0)
                plsc.store_scatter(pend, [pos], v, mask=mask)
                return pending + cnt

            pending = lax.fori_loop(0, CH // 16, step, pending)

            nblk = lax.shift_right_logical(pending, 7)

            def wblk(b, c):
                pltpu.sync_copy(
                    pend.at[pl.ds(b * 128, 128)],
                    lists.at[pl.ds(lbase + (written + b) * 128, 128)])
                return c
            lax.fori_loop(0, nblk, wblk, 0)

            mbase = nblk * 128
            for g in range(8):
                rem = plsc.load_gather(pend, [mbase + g * 16 + iota])
                pend[g * 16:(g + 1) * 16] = rem
            return (lax.bitwise_and(pending, 127), written + nblk)

        def start_chunk(ci, half, sem):
            pltpu.async_copy(packed_ref.at[pl.ds(ci * CH, CH)],
                             ebuf.at[pl.ds(half * CH, CH)], sem)

        def wait_chunk(sem):
            pltpu.make_async_copy(packed_ref.at[pl.ds(0, CH)],
                                  ebuf.at[pl.ds(0, CH)], sem).wait()

        start_chunk(0, 0, sema)

        def pair_body(pp, carry):
            ci0 = pp * 2
            wait_chunk(sema)
            start_chunk(ci0 + 1, 1, semb)
            carry = scan_flush(0, carry)
            wait_chunk(semb)

            @pl.when(ci0 + 2 < NCH)
            def _():
                start_chunk(ci0 + 2, 0, sema)

            return scan_flush(CH, carry)

        pending, written = lax.fori_loop(0, NCH // 2, pair_body,
                                         (jnp.int32(0), jnp.int32(0)))

        def pads(q, c):
            plsc.store_scatter(pend, [pending + q * 16 + iota], sentv)
            return c
        lax.fori_loop(0, 17, pads, 0)
        ntail = lax.shift_right_logical(pending + 143, 7)

        def wtail(b, c):
            pltpu.sync_copy(
                pend.at[pl.ds(b * 128, 128)],
                lists.at[pl.ds(lbase + (written + b) * 128, 128)])
            return c
        lax.fori_loop(0, ntail, wtail, 0)
        written = written + ntail

        for q in range(8):
            pend[q * 16:(q + 1) * 16] = sentv

        def wsent(b, c):
            pltpu.sync_copy(
                pend.at[pl.ds(0, 128)],
                lists.at[pl.ds(lbase + (written + b) * 128, 128)])
            return c
        lax.fori_loop(0, LCH // 128, wsent, 0)

    return k(packed)


def _sc_agg(h_list, lists):
    nps = len(h_list)
    mesh = plsc.VectorSubcoreMesh(core_axis_name="c", subcore_axis_name="s")

    @functools.partial(
        pl.kernel,
        mesh=mesh,
        out_type=[jax.ShapeDtypeStruct((NPAD * 4 * 64,), jnp.int32)] * nps,
        scratch_types=[
            pltpu.VMEM((LCH,), jnp.int32),
            pltpu.VMEM((256,), jnp.int32),
            pltpu.VMEM((256, 64), jnp.int32),
            pltpu.VMEM(((SLOTS + 1) * 64,), jnp.int32),
            pltpu.SemaphoreType.DMA,
            pltpu.SemaphoreType.DMA,
        ],
        **_SC_PARAMS,
    )
    def k(*refs):
        h_refs = refs[:nps]
        lists_ref = refs[nps]
        out_refs = refs[nps + 1:nps + 1 + nps]
        ebuf, idxbuf, gbuf, acc, sema, semb = refs[nps + 1 + nps:]

        wid = lax.axis_index("s") * 2 + lax.axis_index("c")
        lo = wid * SLOTS
        hi = lo + SLOTS
        lbase = wid * ECAP
        iota = lax.iota(jnp.int32, 16)
        colv = [kk * 16 + iota for kk in range(4)]
        sent = lax.shift_left(hi, 16)
        negv = jnp.full((16,), NEGW, jnp.int32)

        for p in range(nps):
            h_hbm = h_refs[p]
            out_hbm = out_refs[p]

            def initb(j, c):
                plsc.store_scatter(acc, [j * 16 + iota], negv)
                return c
            lax.fori_loop(0, (SLOTS + 1) * 4, initb, 0)

            def apply_blk(b, po):
                def gloop(g, c1):
                    grp = plsc.load_gather(ebuf, [b * 128 + g * 16 + iota])
                    slotloc = lax.shift_right_logical(grp, 16) - lo
                    base = po + g * 16
                    for r in range(16):
                        rr = jnp.full((16,), r, jnp.int32)
                        sl64 = slotloc.at[rr].get(
                            mode="promise_in_bounds") * 64
                        row = jnp.full((16,), base + r, jnp.int32)
                        for kk in range(4):
                            msg = plsc.bitcast(
                                plsc.load_gather(gbuf, [row, colv[kk]]),
                                jnp.bfloat16)
                            idxk = sl64 + colv[kk]
                            cur = plsc.bitcast(
                                plsc.load_gather(acc, [idxk]), jnp.bfloat16)
                            plsc.store_scatter(
                                acc, [idxk],
                                plsc.bitcast(jnp.maximum(cur, msg),
                                             jnp.int32))
                    return c1
                lax.fori_loop(0, 8, gloop, 0)

            def build_start(b, hb, sem):
                bv = plsc.load_gather(ebuf, [b * 128 + iota])
                s0 = jnp.sum(jnp.where(iota == 0, bv, 0))
                bstart = s0 != sent

                def bloop(g, c1):
                    grp = plsc.load_gather(ebuf, [b * 128 + g * 16 + iota])
                    plsc.store_scatter(idxbuf, [hb + g * 16 + iota],
                                       lax.bitwise_and(grp, 0xFFFF))
                    return c1
                lax.fori_loop(0, 8, bloop, 0)

                @pl.when(bstart)
                def _():
                    pltpu.async_copy(
                        h_hbm.at[idxbuf.at[pl.ds(hb, 128)]],
                        gbuf.at[pl.ds(hb, 128)], sem)
                return bstart

            def wait_g(sem):
                pltpu.make_async_copy(
                    h_hbm.at[idxbuf.at[pl.ds(0, 128)]],
                    gbuf.at[pl.ds(0, 128)], sem).wait()

            def chunk_body(ci, go):
                running = go > 0

                @pl.when(running)
                def _():
                    pltpu.sync_copy(
                        lists_ref.at[pl.ds(lbase + ci * LCH, LCH)], ebuf)

                v0 = plsc.load_gather(ebuf, [iota])
                vl = plsc.load_gather(ebuf, [LCH - 16 + iota])
                s_first = jnp.sum(jnp.where(iota == 0, v0, 0))
                s_last = jnp.sum(jnp.where(iota == 15, vl, 0))
                process = running & (s_first != sent)
                go_next = running & (s_last != sent)

                def pair(pp, started_odd):
                    b0 = pp * 2
                    bs0 = build_start(b0, 0, sema)

                    @pl.when(started_odd > 0)
                    def _():
                        wait_g(semb)
                        apply_blk(b0 - 1, 128)

                    bs1 = build_start(b0 + 1, 128, semb)

                    @pl.when(bs0)
                    def _():
                        wait_g(sema)
                        apply_blk(b0, 0)

                    return lax.select(bs1, 1, 0)

                npair = lax.select(process, LCH // 256, 0)
                started_odd = lax.fori_loop(0, npair, pair, jnp.int32(0))

                @pl.when(started_odd > 0)
                def _():
                    wait_g(semb)
                    apply_blk(LCH // 128 - 1, 128)

                return lax.select(go_next, 1, 0)

            lax.fori_loop(0, MAXCH, chunk_body, jnp.int32(1))

            pltpu.sync_copy(acc.at[pl.ds(0, SLOTS * 64)],
                            out_hbm.at[pl.ds(wid * SLOTS * 64, SLOTS * 64)])

    return list(k(*h_list, lists))


def kernel(x, edge_index, edge_type, W_ih, W_hh, b_ih, b_hh,
           weights1, bias1, weights2, bias2,
           gamma1, beta1, gamma2, beta2,
           Wself1, bself1, Wself2, bself2, osc):
    Wg = W_ih.T
    bg = (b_ih + b_hh).reshape(1, 4 * D)
    w1c = [weights1[:, :, 64 * c:64 * (c + 1)].transpose(0, 2, 1).reshape(256, 2 * D)
           for c in range(2)]
    b1 = (bself1 + 4.0 * bias1).reshape(1, 2 * D)
    w2c = [weights2[:, :, 64 * c:64 * (c + 1)].transpose(0, 2, 1).reshape(256, D)
           for c in range(4)]
    b2 = (bself2 + 4.0 * bias2).reshape(1, D)

    src2 = edge_index[0].reshape(E // 128, 128)
    dst2 = edge_index[1].reshape(E // 128, 128)
    et2 = edge_type.reshape(E // 128, 128)

    packed = _pack_stage(src2, dst2, et2).reshape(E)
    h1, h1w = _lstm_stage(x, Wg, bg)
    lists = _sc_partition(packed)

    def unpk(a):
        return a.reshape(NPAD, 256)[:N]

    a1 = unpk(_sc_agg([h1w], lists)[0])

    out1, st1 = _mix1_stage(h1, a1, Wself1.T, w1c[0], w1c[1], b1)
    h2, h2w0, h2w1 = _bnrelu_stage(out1, st1, gamma1.reshape(1, 2 * D),
                                   beta1.reshape(1, 2 * D))

    agg2 = _sc_agg([h2w0, h2w1], lists)
    a2 = [unpk(a) for a in agg2]

    out2, st2 = _mix2_stage(h2, a2[0], a2[1], Wself2.T, w2c, b2)
    return _final_stage(out2, st2, gamma2.reshape(1, D), beta2.reshape(1, D))
